# baseline (speedup 1.0000x reference)
import functools

import jax
import jax.numpy as jnp
from jax import lax
from jax.experimental import pallas as pl
from jax.experimental.pallas import tpu as pltpu
from jax.experimental.pallas import tpu_sc as plsc

N = 10000
E = 160000
D = 256
HALF = 128
NB = 10
RB = N // NB
NC, NS, LANES = 2, 16, 16
K_MASK = 1500
K_NOISE = 150
NPAD = 160
CHUNK = 64
NROW = 2560
EPAD = NROW * CHUNK
NCH = NROW // NS
NCHH = NCH // 2
EPW = NCH * CHUNK
ACCN = N + 8
NV = N // LANES
RPW = 624
RTAIL = N - NS * RPW

_MESH = dict(core_axis_name="c", subcore_axis_name="s")


def _split_body(x_ref, o_ref):
    o_ref[...] = x_ref[...]


def _split(x):
    return pl.pallas_call(
        _split_body,
        grid=(NC, NB),
        in_specs=[pl.BlockSpec((RB, HALF), lambda c, i: (i, c))],
        out_specs=pl.BlockSpec((RB, HALF), lambda c, i: (c * NB + i, 0)),
        out_shape=jax.ShapeDtypeStruct((2 * N, HALF), jnp.float32),
    )(x)


def _z1_body(ha, hb, ga, gb, w, b, o):
    z = jnp.concatenate([ha[...] + ga[...], hb[...] + gb[...]], axis=1)
    o[...] = jnp.maximum(
        jnp.dot(z, w[...], preferred_element_type=jnp.float32) + b[...], 0.0)


def _z1(h2, agg2, W1, b1):
    return pl.pallas_call(
        _z1_body,
        grid=(NB,),
        in_specs=[
            pl.BlockSpec((RB, HALF), lambda i: (i, 0)),
            pl.BlockSpec((RB, HALF), lambda i: (i + NB, 0)),
            pl.BlockSpec((RB, HALF), lambda i: (i, 0)),
            pl.BlockSpec((RB, HALF), lambda i: (i + NB, 0)),
            pl.BlockSpec((D, D), lambda i: (0, 0)),
            pl.BlockSpec((1, D), lambda i: (0, 0)),
        ],
        out_specs=pl.BlockSpec((RB, D), lambda i: (i, 0)),
        out_shape=jax.ShapeDtypeStruct((N, D), jnp.float32),
    )(h2, h2, agg2, agg2, W1, b1.reshape(1, D))


def _z2_body(z, w, b, o, *, relu):
    r = jnp.dot(z[...], w[...], preferred_element_type=jnp.float32) + b[...]
    if relu:
        r = jnp.maximum(r, 0.0)
    o[...] = r


def _z2(z1, W2, b2, relu):
    return pl.pallas_call(
        functools.partial(_z2_body, relu=relu),
        grid=(NC, NB),
        in_specs=[
            pl.BlockSpec((RB, D), lambda c, i: (i, 0)),
            pl.BlockSpec((D, HALF), lambda c, i: (0, c)),
            pl.BlockSpec((1, HALF), lambda c, i: (0, c)),
        ],
        out_specs=pl.BlockSpec((RB, HALF), lambda c, i: (c * NB + i, 0)),
        out_shape=jax.ShapeDtypeStruct((2 * N, HALF), jnp.float32),
    )(z1, W2, b2.reshape(1, D))


def _score_body(z, w, b, o):
    s = jax.nn.sigmoid(
        jnp.dot(z[...], w[...], preferred_element_type=jnp.float32) + b[...])
    o[...] = jax.lax.bitcast_convert_type(s, jnp.int32)


def _score(z1, sg_W2, sg_b2):
    return pl.pallas_call(
        _score_body,
        grid=(NB,),
        in_specs=[
            pl.BlockSpec((RB, D), lambda i: (i, 0)),
            pl.BlockSpec((D, 1), lambda i: (0, 0)),
            pl.BlockSpec((1, 1), lambda i: (0, 0)),
        ],
        out_specs=pl.BlockSpec((RB, 1), lambda i: (i, 0)),
        out_shape=jax.ShapeDtypeStruct((N, 1), jnp.int32),
    )(z1, sg_W2, sg_b2.reshape(1, 1))


def _maskx_body(xa, m, mn, tok, o):
    mm = m[...]
    o[...] = xa[...] * (1.0 - mm) + (mm - mn[...]) * tok[...]


def _maskx(x2, m, mn, token):
    return pl.pallas_call(
        _maskx_body,
        grid=(NC, NB),
        in_specs=[
            pl.BlockSpec((RB, HALF), lambda c, i: (c * NB + i, 0)),
            pl.BlockSpec((RB, 1), lambda c, i: (i, 0)),
            pl.BlockSpec((RB, 1), lambda c, i: (i, 0)),
            pl.BlockSpec((1, HALF), lambda c, i: (0, c)),
        ],
        out_specs=pl.BlockSpec((RB, HALF), lambda c, i: (c * NB + i, 0)),
        out_shape=jax.ShapeDtypeStruct((2 * N, HALF), jnp.float32),
    )(x2, m, mn, token)


def _rep_body(ha, hb, w, m, o):
    z = jnp.concatenate([ha[...], hb[...]], axis=1)
    o[...] = jnp.dot(z, w[...], preferred_element_type=jnp.float32) * (1.0 - m[...])


def _rep(h2, e2d_W, m):
    return pl.pallas_call(
        _rep_body,
        grid=(NC, NB),
        in_specs=[
            pl.BlockSpec((RB, HALF), lambda c, i: (i, 0)),
            pl.BlockSpec((RB, HALF), lambda c, i: (i + NB, 0)),
            pl.BlockSpec((D, HALF), lambda c, i: (0, c)),
            pl.BlockSpec((RB, 1), lambda c, i: (i, 0)),
        ],
        out_specs=pl.BlockSpec((RB, HALF), lambda c, i: (c * NB + i, 0)),
        out_shape=jax.ShapeDtypeStruct((2 * N, HALF), jnp.float32),
    )(h2, h2, e2d_W, m)


def _loss_body(ra, rb, xf, m, o):
    r = jnp.concatenate([ra[...], rb[...]], axis=1)
    t = xf[...]
    rr = jnp.sum(r * r, axis=1, keepdims=True)
    tt = jnp.sum(t * t, axis=1, keepdims=True)
    rt = jnp.sum(r * t, axis=1, keepdims=True)
    cos = rt / ((jnp.sqrt(rr) + 1e-8) * (jnp.sqrt(tt) + 1e-8))
    dd = 1.0 - cos
    v = jnp.sum(dd * dd * m[...]) * (1.0 / K_MASK)

    @pl.when(pl.program_id(0) == 0)
    def _():
        o[...] = jnp.zeros((1, 1), jnp.float32)

    o[...] += jnp.reshape(v, (1, 1))


def _loss(recon2, x, m):
    return pl.pallas_call(
        _loss_body,
        grid=(NB,),
        in_specs=[
            pl.BlockSpec((RB, HALF), lambda i: (i, 0)),
            pl.BlockSpec((RB, HALF), lambda i: (i + NB, 0)),
            pl.BlockSpec((RB, D), lambda i: (i, 0)),
            pl.BlockSpec((RB, 1), lambda i: (i, 0)),
        ],
        out_specs=pl.BlockSpec((1, 1), lambda i: (0, 0)),
        out_shape=jax.ShapeDtypeStruct((1, 1), jnp.float32),
    )(recon2, recon2, x, m)


def _segsum(h2, src, dst2, zsrc):
    mesh = plsc.VectorSubcoreMesh(**_MESH)

    @functools.partial(
        pl.kernel, mesh=mesh,
        compiler_params=pltpu.CompilerParams(needs_layout_passes=False),
        out_type=jax.ShapeDtypeStruct((2 * N, HALF), jnp.float32),
        scratch_types=[
            pltpu.VMEM((EPW,), jnp.int32),
            pltpu.VMEM((NCH, CHUNK), jnp.int32),
            pltpu.VMEM((CHUNK, HALF), jnp.float32),
            pltpu.VMEM((CHUNK, HALF), jnp.float32),
            pltpu.VMEM_SHARED((ACCN, HALF), jnp.float32),
            pltpu.SemaphoreType.DMA,
            pltpu.SemaphoreType.DMA,
            pltpu.SemaphoreType.DMA,
            pltpu.SemaphoreType.DMA,
        ],
    )
    def k(h2_hbm, src_hbm, dst2_hbm, zsrc_hbm, out_hbm,
          aidx, didx2, gbuf0, gbuf1, acc, sem0, sem1, ssem0, ssem1):
        c = lax.axis_index("c")
        s = lax.axis_index("s")
        coff = c * N
        r0 = s * RPW
        e0 = s * EPW

        pltpu.sync_copy(src_hbm.at[pl.ds(e0, EPW)], aidx)
        pltpu.sync_copy(dst2_hbm.at[pl.ds(s * NCH, NCH)], didx2)

        pltpu.sync_copy(zsrc_hbm.at[pl.ds(0, RPW)], acc.at[pl.ds(r0, RPW)])

        @pl.when(s == NS - 1)
        def _():
            pltpu.sync_copy(zsrc_hbm.at[pl.ds(0, RTAIL)],
                            acc.at[pl.ds(NS * RPW, RTAIL)])

        def adj(l, carry):
            sl = pl.ds(l * LANES, LANES)
            aidx[sl] = aidx[sl] + coff
            return carry
        lax.fori_loop(0, EPW // LANES, adj, 0)

        plsc.subcore_barrier()

        def start(j, gb, sem):
            sl = aidx.at[pl.ds(j * CHUNK, CHUNK)]
            pltpu.async_copy(h2_hbm.at[sl], gb, sem)

        start(0, gbuf0, sem0)
        start(1, gbuf1, sem1)

        def outer(g, carry):
            j0 = 2 * g
            pltpu.make_async_copy(h2_hbm, gbuf0, sem0).wait()
            pltpu.async_copy(gbuf0, acc.at[didx2.at[j0]], ssem0, add=True)
            pltpu.make_async_copy(h2_hbm, gbuf1, sem1).wait()
            pltpu.async_copy(gbuf1, acc.at[didx2.at[j0 + 1]], ssem1, add=True)
            pltpu.make_async_copy(gbuf0, acc.at[didx2.at[j0]], ssem0).wait()
            start(j0 + 2, gbuf0, sem0)
            pltpu.make_async_copy(gbuf1, acc.at[didx2.at[j0 + 1]],
                                  ssem1).wait()
            start(j0 + 3, gbuf1, sem1)
            return carry
        lax.fori_loop(0, NCHH - 1, outer, 0)

        jf = NCH - 2
        pltpu.make_async_copy(h2_hbm, gbuf0, sem0).wait()
        pltpu.sync_copy(gbuf0, acc.at[didx2.at[jf]], add=True)
        pltpu.make_async_copy(h2_hbm, gbuf1, sem1).wait()
        pltpu.sync_copy(gbuf1, acc.at[didx2.at[jf + 1]], add=True)

        plsc.subcore_barrier()
        pltpu.sync_copy(acc.at[pl.ds(r0, RPW)],
                        out_hbm.at[pl.ds(coff + r0, RPW)])

        @pl.when(s == NS - 1)
        def _():
            pltpu.sync_copy(acc.at[pl.ds(NS * RPW, RTAIL)],
                            out_hbm.at[pl.ds(coff + NS * RPW, RTAIL)])

    return k(h2, src, dst2, zsrc)


def _select(scores):
    mesh = plsc.VectorSubcoreMesh(**_MESH)
    NT = NPAD // LANES

    @functools.partial(
        pl.kernel, mesh=mesh,
        compiler_params=pltpu.CompilerParams(needs_layout_passes=False),
        out_type=(jax.ShapeDtypeStruct((N,), jnp.float32),
                  jax.ShapeDtypeStruct((N,), jnp.float32),
                  jax.ShapeDtypeStruct((NPAD,), jnp.int32)),
        scratch_types=[
            pltpu.VMEM((N,), jnp.int32),
            pltpu.VMEM((N,), jnp.float32),
            pltpu.VMEM((N,), jnp.float32),
            pltpu.VMEM((176,), jnp.int32),
            pltpu.VMEM((176,), jnp.int32),
            pltpu.VMEM((NPAD,), jnp.int32),
            pltpu.SemaphoreType.DMA,
        ],
    )
    def k(sc_hbm, m_hbm, mn_hbm, nd_hbm,
          kbuf, mbuf, mnbuf, tk, ti, nd, sem):
        c = lax.axis_index("c")
        s = lax.axis_index("s")

        @pl.when(jnp.logical_and(c == 0, s == 0))
        def _():
            pltpu.sync_copy(sc_hbm, kbuf)

            one16 = jnp.ones((LANES,), jnp.int32)
            zero16i = jnp.zeros((LANES,), jnp.int32)

            def count_ge(t):
                def cb(j, acc):
                    v = kbuf[pl.ds(j * LANES, LANES)]
                    return acc + jnp.where(v >= t, one16, zero16i)
                accv = lax.fori_loop(0, NV, cb, jnp.zeros((LANES,), jnp.int32))
                return jnp.sum(accv)

            def kth_key(kk):
                def bb(it, lh):
                    lo, hi = lh
                    mid = lo + ((hi - lo) >> 1)
                    big = count_ge(mid) >= kk
                    return (jnp.where(big, mid, lo), jnp.where(big, hi, mid))
                lo, _ = lax.fori_loop(
                    0, 32, bb, (jnp.int32(0), jnp.int32(0x7FFFFFFF)))
                return lo

            t1500 = kth_key(K_MASK)
            t150 = kth_key(K_NOISE)

            zero16 = jnp.zeros((LANES,), jnp.int32)
            for j in range(176 // LANES):
                tk[pl.ds(j * LANES, LANES)] = zero16

            iota16 = lax.iota(jnp.int32, LANES)

            def mb(j, off):
                sl = pl.ds(j * LANES, LANES)
                v = kbuf[sl]
                m1 = v >= t1500
                m2 = v >= t150
                mbuf[sl] = jnp.where(m1, 1.0, 0.0)
                mnbuf[sl] = jnp.where(m2, 1.0, 0.0)
                pc = plsc.cumsum(jnp.where(m2, one16, zero16i))
                pos = off + pc - 1
                ids = j * LANES + iota16
                plsc.store_scatter(tk, [pos], v, mask=m2)
                plsc.store_scatter(ti, [pos], ids, mask=m2)
                return off + jnp.max(pc)
            lax.fori_loop(0, NV, mb, jnp.int32(0))

            lane0 = iota16 == 0

            def rb(i, carry):
                iv = jnp.full((LANES,), i, jnp.int32)
                kv = plsc.load_gather(tk, [iv])

                def cb2(j, a):
                    w = tk[pl.ds(j * LANES, LANES)]
                    return a + jnp.where(w > kv, one16, zero16i)
                av = lax.fori_loop(0, NT, cb2, jnp.zeros((LANES,), jnp.int32))
                rv = jnp.full((LANES,), jnp.sum(av), jnp.int32)
                idv = plsc.load_gather(ti, [iv])
                plsc.store_scatter(nd, [rv], idv, mask=lane0)
                return carry
            lax.fori_loop(0, K_NOISE, rb, 0)

            nd0 = plsc.load_gather(nd, [jnp.zeros((LANES,), jnp.int32)])
            plsc.store_scatter(nd, [K_NOISE + iota16], nd0,
                               mask=iota16 < (NPAD - K_NOISE))

            pltpu.sync_copy(mbuf, m_hbm)
            pltpu.sync_copy(mnbuf, mn_hbm)
            pltpu.sync_copy(nd, nd_hbm)

    return k(scores)


def _noise_patch(base, x2, nsrc, nd):
    mesh = plsc.VectorSubcoreMesh(**_MESH)
    ROWS = 624
    CTAIL = 2 * N - NC * NS * ROWS

    @functools.partial(
        pl.kernel, mesh=mesh,
        compiler_params=pltpu.CompilerParams(needs_layout_passes=False),
        out_type=jax.ShapeDtypeStruct((2 * N, HALF), jnp.float32),
        scratch_types=[
            pltpu.VMEM((ROWS, HALF), jnp.float32),
            pltpu.VMEM((CTAIL, HALF), jnp.float32),
            pltpu.VMEM((NPAD,), jnp.int32),
            pltpu.VMEM((NPAD,), jnp.int32),
            pltpu.VMEM((NPAD, HALF), jnp.float32),
            pltpu.SemaphoreType.DMA,
        ],
    )
    def k(base_hbm, x2_hbm, nsrc_hbm, nd_hbm, out_hbm, buf, tbuf, gi, si,
          nbuf, sem):
        c = lax.axis_index("c")
        s = lax.axis_index("s")
        wid = c * NS + s
        r0 = wid * ROWS
        pltpu.sync_copy(base_hbm.at[pl.ds(r0, ROWS)], buf)
        pltpu.sync_copy(buf, out_hbm.at[pl.ds(r0, ROWS)])

        @pl.when(wid == NC * NS - 1)
        def _():
            t0 = NC * NS * ROWS
            pltpu.sync_copy(base_hbm.at[pl.ds(t0, CTAIL)], tbuf)
            pltpu.sync_copy(tbuf, out_hbm.at[pl.ds(t0, CTAIL)])

        plsc.subcore_barrier()

        @pl.when(s == 0)
        def _():
            coff = c * N
            pltpu.sync_copy(nsrc_hbm, gi)
            pltpu.sync_copy(nd_hbm, si)
            for l in range(NPAD // LANES):
                sl = pl.ds(l * LANES, LANES)
                gi[sl] = gi[sl] + coff
                si[sl] = si[sl] + coff
            pltpu.async_copy(x2_hbm.at[gi], nbuf, sem).wait()
            pltpu.sync_copy(nbuf, out_hbm.at[si])

    return k(base, x2, nsrc, nd)


def kernel(x, edge_index, epoch, max_epoch, sg_W1, sg_b1, sg_W2, sg_b2,
           sg2_W, sg2_b, e1_W1, e1_b1, e1_W2, e1_b2, e2_W1, e2_b1, e2_W2,
           e2_b2, e2d_W, d_W1, d_b1, d_W2, d_b2, enc_mask_token):
    npad = EPAD - E
    src = jnp.concatenate([edge_index[0], jnp.zeros((npad,), jnp.int32)])
    dst = jnp.concatenate([edge_index[1],
                           jnp.full((npad,), N, jnp.int32)]).reshape(
                               NROW, CHUNK)
    zsrc = jnp.zeros((RPW, HALF), jnp.float32)

    x2 = _split(x)

    agg0 = _segsum(x2, src, dst, zsrc)
    z1s = _z1(x2, agg0, sg_W1, sg_b1)
    scores = _score(z1s, sg_W2, sg_b2)

    m, mn, nd = _select(scores.reshape(N))
    m = m.reshape(N, 1)
    mn = mn.reshape(N, 1)

    nsrc = jax.random.randint(jax.random.key(7), (K_NOISE,), 0, N,
                              dtype=jnp.int32)
    nsrc = jnp.concatenate([nsrc, jnp.full((NPAD - K_NOISE,), nsrc[0],
                                           jnp.int32)])

    base = _maskx(x2, m, mn, enc_mask_token)
    out_x2 = _noise_patch(base, x2, nsrc, nd)

    agg1 = _segsum(out_x2, src, dst, zsrc)
    h = _z2(_z1(out_x2, agg1, e1_W1, e1_b1), e1_W2, e1_b2, relu=True)
    agg2 = _segsum(h, src, dst, zsrc)
    h = _z2(_z1(h, agg2, e2_W1, e2_b1), e2_W2, e2_b2, relu=True)

    rep = _rep(h, e2d_W, m)

    agg3 = _segsum(rep, src, dst, zsrc)
    recon = _z2(_z1(rep, agg3, d_W1, d_b1), d_W2, d_b2, relu=False)

    lossv = _loss(recon, x, m)[0, 0]
    ep = jnp.asarray(epoch).astype(jnp.float32)
    mep = jnp.asarray(max_epoch).astype(jnp.float32)
    return lossv + 0.0 * (0.3 * ep / mep)


# speedup vs baseline: 1.0203x; 1.0203x over previous
import functools

import jax
import jax.numpy as jnp
from jax import lax
from jax.experimental import pallas as pl
from jax.experimental.pallas import tpu as pltpu
from jax.experimental.pallas import tpu_sc as plsc

N = 10000
E = 160000
D = 256
HALF = 128
NB = 10
RB = N // NB
NC, NS, LANES = 2, 16, 16
K_MASK = 1500
K_NOISE = 150
NPAD = 160
CHUNK = 128
NROW = 1280
EPAD = NROW * CHUNK
NCH = NROW // NS
NWIN = NCH // 2
HWIN = NWIN // 2
EPW = NCH * CHUNK
ACCN = N + 8
NV = N // LANES
RPW = 624
RTAIL = N - NS * RPW

_MESH = dict(core_axis_name="c", subcore_axis_name="s")


def _split_body(x_ref, o_ref):
    o_ref[...] = x_ref[...]


def _split(x):
    return pl.pallas_call(
        _split_body,
        grid=(NC, NB),
        in_specs=[pl.BlockSpec((RB, HALF), lambda c, i: (i, c))],
        out_specs=pl.BlockSpec((RB, HALF), lambda c, i: (c * NB + i, 0)),
        out_shape=jax.ShapeDtypeStruct((2 * N, HALF), jnp.float32),
    )(x)


def _z1_body(ha, hb, ga, gb, w, b, o):
    z = jnp.concatenate([ha[...] + ga[...], hb[...] + gb[...]], axis=1)
    o[...] = jnp.maximum(
        jnp.dot(z, w[...], preferred_element_type=jnp.float32) + b[...], 0.0)


def _z1(h2, agg2, W1, b1):
    return pl.pallas_call(
        _z1_body,
        grid=(NB,),
        in_specs=[
            pl.BlockSpec((RB, HALF), lambda i: (i, 0)),
            pl.BlockSpec((RB, HALF), lambda i: (i + NB, 0)),
            pl.BlockSpec((RB, HALF), lambda i: (i, 0)),
            pl.BlockSpec((RB, HALF), lambda i: (i + NB, 0)),
            pl.BlockSpec((D, D), lambda i: (0, 0)),
            pl.BlockSpec((1, D), lambda i: (0, 0)),
        ],
        out_specs=pl.BlockSpec((RB, D), lambda i: (i, 0)),
        out_shape=jax.ShapeDtypeStruct((N, D), jnp.float32),
    )(h2, h2, agg2, agg2, W1, b1.reshape(1, D))


def _z2_body(z, w, b, o, *, relu):
    r = jnp.dot(z[...], w[...], preferred_element_type=jnp.float32) + b[...]
    if relu:
        r = jnp.maximum(r, 0.0)
    o[...] = r


def _z2(z1, W2, b2, relu):
    return pl.pallas_call(
        functools.partial(_z2_body, relu=relu),
        grid=(NC, NB),
        in_specs=[
            pl.BlockSpec((RB, D), lambda c, i: (i, 0)),
            pl.BlockSpec((D, HALF), lambda c, i: (0, c)),
            pl.BlockSpec((1, HALF), lambda c, i: (0, c)),
        ],
        out_specs=pl.BlockSpec((RB, HALF), lambda c, i: (c * NB + i, 0)),
        out_shape=jax.ShapeDtypeStruct((2 * N, HALF), jnp.float32),
    )(z1, W2, b2.reshape(1, D))


def _score_body(z, w, b, o):
    s = jax.nn.sigmoid(
        jnp.dot(z[...], w[...], preferred_element_type=jnp.float32) + b[...])
    o[...] = jax.lax.bitcast_convert_type(s, jnp.int32)


def _score(z1, sg_W2, sg_b2):
    return pl.pallas_call(
        _score_body,
        grid=(NB,),
        in_specs=[
            pl.BlockSpec((RB, D), lambda i: (i, 0)),
            pl.BlockSpec((D, 1), lambda i: (0, 0)),
            pl.BlockSpec((1, 1), lambda i: (0, 0)),
        ],
        out_specs=pl.BlockSpec((RB, 1), lambda i: (i, 0)),
        out_shape=jax.ShapeDtypeStruct((N, 1), jnp.int32),
    )(z1, sg_W2, sg_b2.reshape(1, 1))


def _maskx_body(xa, m, mn, tok, o):
    mm = m[...]
    o[...] = xa[...] * (1.0 - mm) + (mm - mn[...]) * tok[...]


def _maskx(x2, m, mn, token):
    return pl.pallas_call(
        _maskx_body,
        grid=(NC, NB),
        in_specs=[
            pl.BlockSpec((RB, HALF), lambda c, i: (c * NB + i, 0)),
            pl.BlockSpec((RB, 1), lambda c, i: (i, 0)),
            pl.BlockSpec((RB, 1), lambda c, i: (i, 0)),
            pl.BlockSpec((1, HALF), lambda c, i: (0, c)),
        ],
        out_specs=pl.BlockSpec((RB, HALF), lambda c, i: (c * NB + i, 0)),
        out_shape=jax.ShapeDtypeStruct((2 * N, HALF), jnp.float32),
    )(x2, m, mn, token)


def _rep_body(ha, hb, w, m, o):
    z = jnp.concatenate([ha[...], hb[...]], axis=1)
    o[...] = jnp.dot(z, w[...], preferred_element_type=jnp.float32) * (1.0 - m[...])


def _rep(h2, e2d_W, m):
    return pl.pallas_call(
        _rep_body,
        grid=(NC, NB),
        in_specs=[
            pl.BlockSpec((RB, HALF), lambda c, i: (i, 0)),
            pl.BlockSpec((RB, HALF), lambda c, i: (i + NB, 0)),
            pl.BlockSpec((D, HALF), lambda c, i: (0, c)),
            pl.BlockSpec((RB, 1), lambda c, i: (i, 0)),
        ],
        out_specs=pl.BlockSpec((RB, HALF), lambda c, i: (c * NB + i, 0)),
        out_shape=jax.ShapeDtypeStruct((2 * N, HALF), jnp.float32),
    )(h2, h2, e2d_W, m)


def _loss_body(ra, rb, xf, m, o):
    r = jnp.concatenate([ra[...], rb[...]], axis=1)
    t = xf[...]
    rr = jnp.sum(r * r, axis=1, keepdims=True)
    tt = jnp.sum(t * t, axis=1, keepdims=True)
    rt = jnp.sum(r * t, axis=1, keepdims=True)
    cos = rt / ((jnp.sqrt(rr) + 1e-8) * (jnp.sqrt(tt) + 1e-8))
    dd = 1.0 - cos
    v = jnp.sum(dd * dd * m[...]) * (1.0 / K_MASK)

    @pl.when(pl.program_id(0) == 0)
    def _():
        o[...] = jnp.zeros((1, 1), jnp.float32)

    o[...] += jnp.reshape(v, (1, 1))


def _loss(recon2, x, m):
    return pl.pallas_call(
        _loss_body,
        grid=(NB,),
        in_specs=[
            pl.BlockSpec((RB, HALF), lambda i: (i, 0)),
            pl.BlockSpec((RB, HALF), lambda i: (i + NB, 0)),
            pl.BlockSpec((RB, D), lambda i: (i, 0)),
            pl.BlockSpec((RB, 1), lambda i: (i, 0)),
        ],
        out_specs=pl.BlockSpec((1, 1), lambda i: (0, 0)),
        out_shape=jax.ShapeDtypeStruct((1, 1), jnp.float32),
    )(recon2, recon2, x, m)


def _segsum(h2, src3, dst2, zsrc):
    mesh = plsc.VectorSubcoreMesh(**_MESH)

    @functools.partial(
        pl.kernel, mesh=mesh,
        compiler_params=pltpu.CompilerParams(needs_layout_passes=False),
        out_type=jax.ShapeDtypeStruct((2 * N, HALF), jnp.float32),
        scratch_types=[
            pltpu.VMEM((NCH, CHUNK), jnp.int32),
            pltpu.VMEM((NCH // 2, CHUNK), jnp.int32),
            pltpu.VMEM((CHUNK, HALF), jnp.float32),
            pltpu.VMEM((CHUNK, HALF), jnp.float32),
            pltpu.VMEM_SHARED((ACCN, HALF), jnp.float32),
            pltpu.SemaphoreType.DMA,
            pltpu.SemaphoreType.DMA,
        ],
    )
    def k(h2_hbm, src3_hbm, dst2_hbm, zsrc_hbm, out_hbm,
          aidx2, didx2, bufa, bufb, acc, sema, semb):
        c = lax.axis_index("c")
        s = lax.axis_index("s")
        coff = c * N
        r0 = s * RPW

        pltpu.sync_copy(src3_hbm.at[pl.ds(c * NROW + s * NCH, NCH)], aidx2)
        pltpu.sync_copy(dst2_hbm.at[pl.ds(s * NCH, NCH // 2)], didx2)

        pltpu.sync_copy(zsrc_hbm.at[pl.ds(0, RPW)], acc.at[pl.ds(r0, RPW)])

        @pl.when(s == NS - 1)
        def _():
            pltpu.sync_copy(zsrc_hbm.at[pl.ds(0, RTAIL)],
                            acc.at[pl.ds(NS * RPW, RTAIL)])

        plsc.subcore_barrier()

        def win(g, carry):
            @pl.when(g == HWIN)
            def _():
                pltpu.sync_copy(
                    dst2_hbm.at[pl.ds(s * NCH + NCH // 2, NCH // 2)], didx2)

            j0 = 2 * g
            rd = j0 - jnp.where(g >= HWIN, NCH // 2, 0)
            da = pltpu.async_copy(h2_hbm.at[aidx2.at[j0]], bufa, sema)
            db = pltpu.async_copy(h2_hbm.at[aidx2.at[j0 + 1]], bufb, semb)
            da.wait()
            pltpu.sync_copy(bufa, acc.at[didx2.at[rd]], add=True)
            db.wait()
            pltpu.sync_copy(bufb, acc.at[didx2.at[rd + 1]], add=True)
            return carry
        lax.fori_loop(0, NWIN, win, 0)

        plsc.subcore_barrier()
        pltpu.sync_copy(acc.at[pl.ds(r0, RPW)],
                        out_hbm.at[pl.ds(coff + r0, RPW)])

        @pl.when(s == NS - 1)
        def _():
            pltpu.sync_copy(acc.at[pl.ds(NS * RPW, RTAIL)],
                            out_hbm.at[pl.ds(coff + NS * RPW, RTAIL)])

    return k(h2, src3, dst2, zsrc)


def _select(scores):
    mesh = plsc.VectorSubcoreMesh(**_MESH)
    NT = NPAD // LANES

    @functools.partial(
        pl.kernel, mesh=mesh,
        compiler_params=pltpu.CompilerParams(needs_layout_passes=False),
        out_type=(jax.ShapeDtypeStruct((N,), jnp.float32),
                  jax.ShapeDtypeStruct((N,), jnp.float32),
                  jax.ShapeDtypeStruct((NPAD,), jnp.int32)),
        scratch_types=[
            pltpu.VMEM((N,), jnp.int32),
            pltpu.VMEM((N,), jnp.float32),
            pltpu.VMEM((N,), jnp.float32),
            pltpu.VMEM((176,), jnp.int32),
            pltpu.VMEM((176,), jnp.int32),
            pltpu.VMEM((NPAD,), jnp.int32),
            pltpu.SemaphoreType.DMA,
        ],
    )
    def k(sc_hbm, m_hbm, mn_hbm, nd_hbm,
          kbuf, mbuf, mnbuf, tk, ti, nd, sem):
        c = lax.axis_index("c")
        s = lax.axis_index("s")

        @pl.when(jnp.logical_and(c == 0, s == 0))
        def _():
            pltpu.sync_copy(sc_hbm, kbuf)

            one16 = jnp.ones((LANES,), jnp.int32)
            zero16i = jnp.zeros((LANES,), jnp.int32)

            def count_ge(t):
                def cb(j, acc):
                    v = kbuf[pl.ds(j * LANES, LANES)]
                    return acc + jnp.where(v >= t, one16, zero16i)
                accv = lax.fori_loop(0, NV, cb, jnp.zeros((LANES,), jnp.int32))
                return jnp.sum(accv)

            def kth_key(kk):
                def bb(it, lh):
                    lo, hi = lh
                    mid = lo + ((hi - lo) >> 1)
                    big = count_ge(mid) >= kk
                    return (jnp.where(big, mid, lo), jnp.where(big, hi, mid))
                lo, _ = lax.fori_loop(
                    0, 32, bb, (jnp.int32(0), jnp.int32(0x7FFFFFFF)))
                return lo

            t1500 = kth_key(K_MASK)
            t150 = kth_key(K_NOISE)

            zero16 = jnp.zeros((LANES,), jnp.int32)
            for j in range(176 // LANES):
                tk[pl.ds(j * LANES, LANES)] = zero16

            iota16 = lax.iota(jnp.int32, LANES)

            def mb(j, off):
                sl = pl.ds(j * LANES, LANES)
                v = kbuf[sl]
                m1 = v >= t1500
                m2 = v >= t150
                mbuf[sl] = jnp.where(m1, 1.0, 0.0)
                mnbuf[sl] = jnp.where(m2, 1.0, 0.0)
                pc = plsc.cumsum(jnp.where(m2, one16, zero16i))
                pos = off + pc - 1
                ids = j * LANES + iota16
                plsc.store_scatter(tk, [pos], v, mask=m2)
                plsc.store_scatter(ti, [pos], ids, mask=m2)
                return off + jnp.max(pc)
            lax.fori_loop(0, NV, mb, jnp.int32(0))

            lane0 = iota16 == 0

            def rb(i, carry):
                iv = jnp.full((LANES,), i, jnp.int32)
                kv = plsc.load_gather(tk, [iv])

                def cb2(j, a):
                    w = tk[pl.ds(j * LANES, LANES)]
                    return a + jnp.where(w > kv, one16, zero16i)
                av = lax.fori_loop(0, NT, cb2, jnp.zeros((LANES,), jnp.int32))
                rv = jnp.full((LANES,), jnp.sum(av), jnp.int32)
                idv = plsc.load_gather(ti, [iv])
                plsc.store_scatter(nd, [rv], idv, mask=lane0)
                return carry
            lax.fori_loop(0, K_NOISE, rb, 0)

            nd0 = plsc.load_gather(nd, [jnp.zeros((LANES,), jnp.int32)])
            plsc.store_scatter(nd, [K_NOISE + iota16], nd0,
                               mask=iota16 < (NPAD - K_NOISE))

            pltpu.sync_copy(mbuf, m_hbm)
            pltpu.sync_copy(mnbuf, mn_hbm)
            pltpu.sync_copy(nd, nd_hbm)

    return k(scores)


def _noise_patch(base, x2, nsrc, nd):
    mesh = plsc.VectorSubcoreMesh(**_MESH)
    ROWS = 624
    CTAIL = 2 * N - NC * NS * ROWS

    @functools.partial(
        pl.kernel, mesh=mesh,
        compiler_params=pltpu.CompilerParams(needs_layout_passes=False),
        out_type=jax.ShapeDtypeStruct((2 * N, HALF), jnp.float32),
        scratch_types=[
            pltpu.VMEM((ROWS, HALF), jnp.float32),
            pltpu.VMEM((CTAIL, HALF), jnp.float32),
            pltpu.VMEM((NPAD,), jnp.int32),
            pltpu.VMEM((NPAD,), jnp.int32),
            pltpu.VMEM((NPAD, HALF), jnp.float32),
            pltpu.SemaphoreType.DMA,
        ],
    )
    def k(base_hbm, x2_hbm, nsrc_hbm, nd_hbm, out_hbm, buf, tbuf, gi, si,
          nbuf, sem):
        c = lax.axis_index("c")
        s = lax.axis_index("s")
        wid = c * NS + s
        r0 = wid * ROWS
        pltpu.sync_copy(base_hbm.at[pl.ds(r0, ROWS)], buf)
        pltpu.sync_copy(buf, out_hbm.at[pl.ds(r0, ROWS)])

        @pl.when(wid == NC * NS - 1)
        def _():
            t0 = NC * NS * ROWS
            pltpu.sync_copy(base_hbm.at[pl.ds(t0, CTAIL)], tbuf)
            pltpu.sync_copy(tbuf, out_hbm.at[pl.ds(t0, CTAIL)])

        plsc.subcore_barrier()

        @pl.when(s == 0)
        def _():
            coff = c * N
            pltpu.sync_copy(nsrc_hbm, gi)
            pltpu.sync_copy(nd_hbm, si)
            for l in range(NPAD // LANES):
                sl = pl.ds(l * LANES, LANES)
                gi[sl] = gi[sl] + coff
                si[sl] = si[sl] + coff
            pltpu.async_copy(x2_hbm.at[gi], nbuf, sem).wait()
            pltpu.sync_copy(nbuf, out_hbm.at[si])

    return k(base, x2, nsrc, nd)


def kernel(x, edge_index, epoch, max_epoch, sg_W1, sg_b1, sg_W2, sg_b2,
           sg2_W, sg2_b, e1_W1, e1_b1, e1_W2, e1_b2, e2_W1, e2_b1, e2_W2,
           e2_b2, e2d_W, d_W1, d_b1, d_W2, d_b2, enc_mask_token):
    npad = EPAD - E
    srcp = jnp.concatenate([edge_index[0], jnp.zeros((npad,), jnp.int32)])
    src = jnp.concatenate([srcp, srcp + N]).reshape(2 * NROW, CHUNK)
    dst = jnp.concatenate([edge_index[1],
                           jnp.full((npad,), N, jnp.int32)]).reshape(
                               NROW, CHUNK)
    zsrc = jnp.zeros((RPW, HALF), jnp.float32)

    x2 = _split(x)

    agg0 = _segsum(x2, src, dst, zsrc)
    z1s = _z1(x2, agg0, sg_W1, sg_b1)
    scores = _score(z1s, sg_W2, sg_b2)

    m, mn, nd = _select(scores.reshape(N))
    m = m.reshape(N, 1)
    mn = mn.reshape(N, 1)

    nsrc = jax.random.randint(jax.random.key(7), (K_NOISE,), 0, N,
                              dtype=jnp.int32)
    nsrc = jnp.concatenate([nsrc, jnp.full((NPAD - K_NOISE,), nsrc[0],
                                           jnp.int32)])

    base = _maskx(x2, m, mn, enc_mask_token)
    out_x2 = _noise_patch(base, x2, nsrc, nd)

    agg1 = _segsum(out_x2, src, dst, zsrc)
    h = _z2(_z1(out_x2, agg1, e1_W1, e1_b1), e1_W2, e1_b2, relu=True)
    agg2 = _segsum(h, src, dst, zsrc)
    h = _z2(_z1(h, agg2, e2_W1, e2_b1), e2_W2, e2_b2, relu=True)

    rep = _rep(h, e2d_W, m)

    agg3 = _segsum(rep, src, dst, zsrc)
    recon = _z2(_z1(rep, agg3, d_W1, d_b1), d_W2, d_b2, relu=False)

    lossv = _loss(recon, x, m)[0, 0]
    ep = jnp.asarray(epoch).astype(jnp.float32)
    mep = jnp.asarray(max_epoch).astype(jnp.float32)
    return lossv + 0.0 * (0.3 * ep / mep)


# speedup vs baseline: 1.3634x; 1.3364x over previous
import functools

import jax
import jax.numpy as jnp
from jax import lax
from jax.experimental import pallas as pl
from jax.experimental.pallas import tpu as pltpu
from jax.experimental.pallas import tpu_sc as plsc

N = 10000
E = 160000
D = 256
HALF = 128
NB = 10
RB = N // NB
NC, NS, LANES = 2, 16, 16
K_MASK = 1500
K_NOISE = 150
NPAD = 160
EPW = E // NS
CHUNK = 128
NFULL = EPW // CHUNK
TAIL = EPW - NFULL * CHUNK
NV = N // LANES
RPW = 624
RTAIL = N - NS * RPW

_MESH = dict(core_axis_name="c", subcore_axis_name="s")


def _split_body(x_ref, o_ref):
    o_ref[...] = x_ref[...]


def _split(x):
    return pl.pallas_call(
        _split_body,
        grid=(NC, NB),
        in_specs=[pl.BlockSpec((RB, HALF), lambda c, i: (i, c))],
        out_specs=pl.BlockSpec((RB, HALF), lambda c, i: (c * NB + i, 0)),
        out_shape=jax.ShapeDtypeStruct((2 * N, HALF), jnp.float32),
    )(x)


def _z1_body(ha, hb, ga, gb, w, b, o):
    z = jnp.concatenate([ha[...] + ga[...], hb[...] + gb[...]], axis=1)
    o[...] = jnp.maximum(
        jnp.dot(z, w[...], preferred_element_type=jnp.float32) + b[...], 0.0)


def _z1(h2, agg2, W1, b1):
    return pl.pallas_call(
        _z1_body,
        grid=(NB,),
        in_specs=[
            pl.BlockSpec((RB, HALF), lambda i: (i, 0)),
            pl.BlockSpec((RB, HALF), lambda i: (i + NB, 0)),
            pl.BlockSpec((RB, HALF), lambda i: (i, 0)),
            pl.BlockSpec((RB, HALF), lambda i: (i + NB, 0)),
            pl.BlockSpec((D, D), lambda i: (0, 0)),
            pl.BlockSpec((1, D), lambda i: (0, 0)),
        ],
        out_specs=pl.BlockSpec((RB, D), lambda i: (i, 0)),
        out_shape=jax.ShapeDtypeStruct((N, D), jnp.float32),
    )(h2, h2, agg2, agg2, W1, b1.reshape(1, D))


def _z2_body(z, w, b, o, *, relu):
    r = jnp.dot(z[...], w[...], preferred_element_type=jnp.float32) + b[...]
    if relu:
        r = jnp.maximum(r, 0.0)
    o[...] = r


def _z2(z1, W2, b2, relu):
    return pl.pallas_call(
        functools.partial(_z2_body, relu=relu),
        grid=(NC, NB),
        in_specs=[
            pl.BlockSpec((RB, D), lambda c, i: (i, 0)),
            pl.BlockSpec((D, HALF), lambda c, i: (0, c)),
            pl.BlockSpec((1, HALF), lambda c, i: (0, c)),
        ],
        out_specs=pl.BlockSpec((RB, HALF), lambda c, i: (c * NB + i, 0)),
        out_shape=jax.ShapeDtypeStruct((2 * N, HALF), jnp.float32),
    )(z1, W2, b2.reshape(1, D))


def _score_body(ha, hb, ga, gb, w1, b1, w2, b2, o):
    z = jnp.concatenate([ha[...] + ga[...], hb[...] + gb[...]], axis=1)
    z1 = jnp.maximum(
        jnp.dot(z, w1[...], preferred_element_type=jnp.float32) + b1[...],
        0.0)
    s = jax.nn.sigmoid(
        jnp.dot(z1, w2[...], preferred_element_type=jnp.float32) + b2[...])
    o[...] = jax.lax.bitcast_convert_type(s, jnp.int32)


def _score(h2, agg2, sg_W1, sg_b1, sg_W2, sg_b2):
    return pl.pallas_call(
        _score_body,
        grid=(NB,),
        in_specs=[
            pl.BlockSpec((RB, HALF), lambda i: (i, 0)),
            pl.BlockSpec((RB, HALF), lambda i: (i + NB, 0)),
            pl.BlockSpec((RB, HALF), lambda i: (i, 0)),
            pl.BlockSpec((RB, HALF), lambda i: (i + NB, 0)),
            pl.BlockSpec((D, D), lambda i: (0, 0)),
            pl.BlockSpec((1, D), lambda i: (0, 0)),
            pl.BlockSpec((D, 1), lambda i: (0, 0)),
            pl.BlockSpec((1, 1), lambda i: (0, 0)),
        ],
        out_specs=pl.BlockSpec((RB, 1), lambda i: (i, 0)),
        out_shape=jax.ShapeDtypeStruct((N, 1), jnp.int32),
    )(h2, h2, agg2, agg2, sg_W1, sg_b1.reshape(1, D), sg_W2,
      sg_b2.reshape(1, 1))


def _maskx_body(xa, m, mn, tok, o):
    mm = m[...]
    o[...] = xa[...] * (1.0 - mm) + (mm - mn[...]) * tok[...]


def _maskx(x2, m, mn, token):
    return pl.pallas_call(
        _maskx_body,
        grid=(NC, NB),
        in_specs=[
            pl.BlockSpec((RB, HALF), lambda c, i: (c * NB + i, 0)),
            pl.BlockSpec((RB, 1), lambda c, i: (i, 0)),
            pl.BlockSpec((RB, 1), lambda c, i: (i, 0)),
            pl.BlockSpec((1, HALF), lambda c, i: (0, c)),
        ],
        out_specs=pl.BlockSpec((RB, HALF), lambda c, i: (c * NB + i, 0)),
        out_shape=jax.ShapeDtypeStruct((2 * N, HALF), jnp.float32),
    )(x2, m, mn, token)


def _rep_body(ha, hb, w, m, o):
    z = jnp.concatenate([ha[...], hb[...]], axis=1)
    o[...] = jnp.dot(z, w[...], preferred_element_type=jnp.float32) * (1.0 - m[...])


def _rep(h2, e2d_W, m):
    return pl.pallas_call(
        _rep_body,
        grid=(NC, NB),
        in_specs=[
            pl.BlockSpec((RB, HALF), lambda c, i: (i, 0)),
            pl.BlockSpec((RB, HALF), lambda c, i: (i + NB, 0)),
            pl.BlockSpec((D, HALF), lambda c, i: (0, c)),
            pl.BlockSpec((RB, 1), lambda c, i: (i, 0)),
        ],
        out_specs=pl.BlockSpec((RB, HALF), lambda c, i: (c * NB + i, 0)),
        out_shape=jax.ShapeDtypeStruct((2 * N, HALF), jnp.float32),
    )(h2, h2, e2d_W, m)


def _loss_body(ra, rb, xf, m, o):
    r = jnp.concatenate([ra[...], rb[...]], axis=1)
    t = xf[...]
    rr = jnp.sum(r * r, axis=1, keepdims=True)
    tt = jnp.sum(t * t, axis=1, keepdims=True)
    rt = jnp.sum(r * t, axis=1, keepdims=True)
    cos = rt / ((jnp.sqrt(rr) + 1e-8) * (jnp.sqrt(tt) + 1e-8))
    dd = 1.0 - cos
    v = jnp.sum(dd * dd * m[...]) * (1.0 / K_MASK)

    @pl.when(pl.program_id(0) == 0)
    def _():
        o[...] = jnp.zeros((1, 1), jnp.float32)

    o[...] += jnp.reshape(v, (1, 1))


def _loss(recon2, x, m):
    return pl.pallas_call(
        _loss_body,
        grid=(NB,),
        in_specs=[
            pl.BlockSpec((RB, HALF), lambda i: (i, 0)),
            pl.BlockSpec((RB, HALF), lambda i: (i + NB, 0)),
            pl.BlockSpec((RB, D), lambda i: (i, 0)),
            pl.BlockSpec((RB, 1), lambda i: (i, 0)),
        ],
        out_specs=pl.BlockSpec((1, 1), lambda i: (0, 0)),
        out_shape=jax.ShapeDtypeStruct((1, 1), jnp.float32),
    )(recon2, recon2, x, m)


def _segsum(h2, src, dst, zsrc):
    mesh = plsc.VectorSubcoreMesh(**_MESH)

    @functools.partial(
        pl.kernel, mesh=mesh,
        compiler_params=pltpu.CompilerParams(needs_layout_passes=False),
        out_type=jax.ShapeDtypeStruct((2 * N, HALF), jnp.float32),
        scratch_types=[
            pltpu.VMEM((CHUNK,), jnp.int32),
            pltpu.VMEM((CHUNK,), jnp.int32),
            pltpu.VMEM((CHUNK,), jnp.int32),
            pltpu.VMEM((CHUNK, HALF), jnp.float32),
            pltpu.VMEM((TAIL,), jnp.int32),
            pltpu.VMEM((TAIL,), jnp.int32),
            pltpu.VMEM((TAIL,), jnp.int32),
            pltpu.VMEM((TAIL, HALF), jnp.float32),
            pltpu.VMEM_SHARED((N, HALF), jnp.float32),
            pltpu.SemaphoreType.DMA,
        ],
    )
    def k(h2_hbm, src_hbm, dst_hbm, zsrc_hbm, out_hbm,
          sidx, aidx, didx, gbuf, tsidx, taidx, tdidx, tgbuf, acc, sem):
        c = lax.axis_index("c")
        s = lax.axis_index("s")
        coff = c * N
        r0 = s * RPW

        pltpu.sync_copy(zsrc_hbm.at[pl.ds(0, RPW)], acc.at[pl.ds(r0, RPW)])

        @pl.when(s == NS - 1)
        def _():
            pltpu.sync_copy(zsrc_hbm.at[pl.ds(0, RTAIL)],
                            acc.at[pl.ds(NS * RPW, RTAIL)])

        plsc.subcore_barrier()

        e0 = s * EPW

        def do_chunk(eoff, cnt, si, ai, di, gb):
            pltpu.sync_copy(src_hbm.at[pl.ds(eoff, cnt)], si)
            pltpu.sync_copy(dst_hbm.at[pl.ds(eoff, cnt)], di)
            for l in range(cnt // LANES):
                sl = pl.ds(l * LANES, LANES)
                ai[sl] = si[sl] + coff
            pltpu.async_copy(h2_hbm.at[ai], gb, sem).wait()
            pltpu.sync_copy(gb, acc.at[di], add=True)

        def body(j, carry):
            do_chunk(e0 + j * CHUNK, CHUNK, sidx, aidx, didx, gbuf)
            return carry

        lax.fori_loop(0, NFULL, body, 0)
        if TAIL:
            do_chunk(e0 + NFULL * CHUNK, TAIL, tsidx, taidx, tdidx, tgbuf)

        plsc.subcore_barrier()
        pltpu.sync_copy(acc.at[pl.ds(r0, RPW)],
                        out_hbm.at[pl.ds(coff + r0, RPW)])

        @pl.when(s == NS - 1)
        def _():
            pltpu.sync_copy(acc.at[pl.ds(NS * RPW, RTAIL)],
                            out_hbm.at[pl.ds(coff + NS * RPW, RTAIL)])

    return k(h2, src, dst, zsrc)


def _select(scores):
    mesh = plsc.VectorSubcoreMesh(**_MESH)
    NT = NPAD // LANES

    @functools.partial(
        pl.kernel, mesh=mesh,
        compiler_params=pltpu.CompilerParams(needs_layout_passes=False),
        out_type=(jax.ShapeDtypeStruct((N,), jnp.float32),
                  jax.ShapeDtypeStruct((N,), jnp.float32),
                  jax.ShapeDtypeStruct((NPAD,), jnp.int32)),
        scratch_types=[
            pltpu.VMEM((N,), jnp.int32),
            pltpu.VMEM((N,), jnp.float32),
            pltpu.VMEM((N,), jnp.float32),
            pltpu.VMEM((176,), jnp.int32),
            pltpu.VMEM((176,), jnp.int32),
            pltpu.VMEM((NPAD,), jnp.int32),
            pltpu.VMEM((1504,), jnp.int32),
            pltpu.SemaphoreType.DMA,
        ],
    )
    def k(sc_hbm, m_hbm, mn_hbm, nd_hbm,
          kbuf, mbuf, mnbuf, tk, ti, nd, t15, sem):
        c = lax.axis_index("c")
        s = lax.axis_index("s")

        @pl.when(jnp.logical_and(c == 0, s == 0))
        def _():
            pltpu.sync_copy(sc_hbm, kbuf)

            one16 = jnp.ones((LANES,), jnp.int32)
            zero16i = jnp.zeros((LANES,), jnp.int32)

            def count_ge(t):
                def cb(j, acc):
                    v = kbuf[pl.ds(j * LANES, LANES)]
                    return acc + jnp.where(v >= t, one16, zero16i)
                accv = lax.fori_loop(0, NV, cb, jnp.zeros((LANES,), jnp.int32))
                return jnp.sum(accv)

            def bisect(kk, counter):
                def bb(it, lh):
                    lo, hi = lh
                    mid = lo + ((hi - lo) >> 1)
                    big = counter(mid) >= kk
                    return (jnp.where(big, mid, lo), jnp.where(big, hi, mid))
                lo, _ = lax.fori_loop(
                    0, 32, bb, (jnp.int32(0), jnp.int32(0x7FFFFFFF)))
                return lo

            t1500 = bisect(K_MASK, count_ge)

            zero16 = jnp.zeros((LANES,), jnp.int32)
            NV15 = 1504 // LANES

            def zt(j, carry):
                t15[pl.ds(j * LANES, LANES)] = zero16
                return carry
            lax.fori_loop(0, NV15, zt, 0)

            def comp(j, off):
                v = kbuf[pl.ds(j * LANES, LANES)]
                m1 = v >= t1500
                pc = plsc.cumsum(jnp.where(m1, one16, zero16i))
                plsc.store_scatter(t15, [off + pc - 1], v, mask=m1)
                return off + jnp.max(pc)
            lax.fori_loop(0, NV, comp, jnp.int32(0))

            def count_ge15(t):
                def cb(j, acc):
                    v = t15[pl.ds(j * LANES, LANES)]
                    return acc + jnp.where(v >= t, one16, zero16i)
                accv = lax.fori_loop(0, NV15, cb,
                                     jnp.zeros((LANES,), jnp.int32))
                return jnp.sum(accv)

            t150 = bisect(K_NOISE, count_ge15)
            for j in range(176 // LANES):
                tk[pl.ds(j * LANES, LANES)] = zero16

            iota16 = lax.iota(jnp.int32, LANES)

            def mb(j, off):
                sl = pl.ds(j * LANES, LANES)
                v = kbuf[sl]
                m1 = v >= t1500
                m2 = v >= t150
                mbuf[sl] = jnp.where(m1, 1.0, 0.0)
                mnbuf[sl] = jnp.where(m2, 1.0, 0.0)
                pc = plsc.cumsum(jnp.where(m2, one16, zero16i))
                pos = off + pc - 1
                ids = j * LANES + iota16
                plsc.store_scatter(tk, [pos], v, mask=m2)
                plsc.store_scatter(ti, [pos], ids, mask=m2)
                return off + jnp.max(pc)
            lax.fori_loop(0, NV, mb, jnp.int32(0))

            lane0 = iota16 == 0

            def rb(i, carry):
                iv = jnp.full((LANES,), i, jnp.int32)
                kv = plsc.load_gather(tk, [iv])

                def cb2(j, a):
                    w = tk[pl.ds(j * LANES, LANES)]
                    return a + jnp.where(w > kv, one16, zero16i)
                av = lax.fori_loop(0, NT, cb2, jnp.zeros((LANES,), jnp.int32))
                rv = jnp.full((LANES,), jnp.sum(av), jnp.int32)
                idv = plsc.load_gather(ti, [iv])
                plsc.store_scatter(nd, [rv], idv, mask=lane0)
                return carry
            lax.fori_loop(0, K_NOISE, rb, 0)

            nd0 = plsc.load_gather(nd, [jnp.zeros((LANES,), jnp.int32)])
            plsc.store_scatter(nd, [K_NOISE + iota16], nd0,
                               mask=iota16 < (NPAD - K_NOISE))

            pltpu.sync_copy(mbuf, m_hbm)
            pltpu.sync_copy(mnbuf, mn_hbm)
            pltpu.sync_copy(nd, nd_hbm)

    return k(scores)


def _noise_patch(base, x2, nsrc, nd):
    mesh = plsc.VectorSubcoreMesh(**_MESH)
    ROWS = 624
    CTAIL = 2 * N - NC * NS * ROWS

    @functools.partial(
        pl.kernel, mesh=mesh,
        compiler_params=pltpu.CompilerParams(needs_layout_passes=False),
        out_type=jax.ShapeDtypeStruct((2 * N, HALF), jnp.float32),
        scratch_types=[
            pltpu.VMEM((ROWS, HALF), jnp.float32),
            pltpu.VMEM((CTAIL, HALF), jnp.float32),
            pltpu.VMEM((NPAD,), jnp.int32),
            pltpu.VMEM((NPAD,), jnp.int32),
            pltpu.VMEM((NPAD, HALF), jnp.float32),
            pltpu.SemaphoreType.DMA,
        ],
    )
    def k(base_hbm, x2_hbm, nsrc_hbm, nd_hbm, out_hbm, buf, tbuf, gi, si,
          nbuf, sem):
        c = lax.axis_index("c")
        s = lax.axis_index("s")
        wid = c * NS + s
        r0 = wid * ROWS
        pltpu.sync_copy(base_hbm.at[pl.ds(r0, ROWS)], buf)
        pltpu.sync_copy(buf, out_hbm.at[pl.ds(r0, ROWS)])

        @pl.when(wid == NC * NS - 1)
        def _():
            t0 = NC * NS * ROWS
            pltpu.sync_copy(base_hbm.at[pl.ds(t0, CTAIL)], tbuf)
            pltpu.sync_copy(tbuf, out_hbm.at[pl.ds(t0, CTAIL)])

        plsc.subcore_barrier()

        @pl.when(s == 0)
        def _():
            coff = c * N
            pltpu.sync_copy(nsrc_hbm, gi)
            pltpu.sync_copy(nd_hbm, si)
            for l in range(NPAD // LANES):
                sl = pl.ds(l * LANES, LANES)
                gi[sl] = gi[sl] + coff
                si[sl] = si[sl] + coff
            pltpu.async_copy(x2_hbm.at[gi], nbuf, sem).wait()
            pltpu.sync_copy(nbuf, out_hbm.at[si])

    return k(base, x2, nsrc, nd)


def kernel(x, edge_index, epoch, max_epoch, sg_W1, sg_b1, sg_W2, sg_b2,
           sg2_W, sg2_b, e1_W1, e1_b1, e1_W2, e1_b2, e2_W1, e2_b1, e2_W2,
           e2_b2, e2d_W, d_W1, d_b1, d_W2, d_b2, enc_mask_token):
    src = edge_index[0]
    dst = edge_index[1]
    zsrc = jnp.zeros((RPW, HALF), jnp.float32)

    x2 = _split(x)

    agg0 = _segsum(x2, src, dst, zsrc)
    scores = _score(x2, agg0, sg_W1, sg_b1, sg_W2, sg_b2)

    m, mn, nd = _select(scores.reshape(N))
    m = m.reshape(N, 1)
    mn = mn.reshape(N, 1)

    nsrc = jax.random.randint(jax.random.key(7), (K_NOISE,), 0, N,
                              dtype=jnp.int32)
    nsrc = jnp.concatenate([nsrc, jnp.full((NPAD - K_NOISE,), nsrc[0],
                                           jnp.int32)])

    base = _maskx(x2, m, mn, enc_mask_token)
    out_x2 = _noise_patch(base, x2, nsrc, nd)

    agg1 = _segsum(out_x2, src, dst, zsrc)
    h = _z2(_z1(out_x2, agg1, e1_W1, e1_b1), e1_W2, e1_b2, relu=True)
    agg2 = _segsum(h, src, dst, zsrc)
    h = _z2(_z1(h, agg2, e2_W1, e2_b1), e2_W2, e2_b2, relu=True)

    rep = _rep(h, e2d_W, m)

    agg3 = _segsum(rep, src, dst, zsrc)
    recon = _z2(_z1(rep, agg3, d_W1, d_b1), d_W2, d_b2, relu=False)

    lossv = _loss(recon, x, m)[0, 0]
    ep = jnp.asarray(epoch).astype(jnp.float32)
    mep = jnp.asarray(max_epoch).astype(jnp.float32)
    return lossv + 0.0 * (0.3 * ep / mep)


# speedup vs baseline: 1.7224x; 1.2633x over previous
import functools

import jax
import jax.numpy as jnp
from jax import lax
from jax.experimental import pallas as pl
from jax.experimental.pallas import tpu as pltpu
from jax.experimental.pallas import tpu_sc as plsc

N = 10000
E = 160000
D = 256
HALF = 128
NB = 10
RB = N // NB
NC, NS, LANES = 2, 16, 16
K_MASK = 1500
K_NOISE = 150
NPAD = 160
EPW = E // NS
CHUNK = 128
NFULL = EPW // CHUNK
TAIL = EPW - NFULL * CHUNK
NV = N // LANES
RPW = 624
RTAIL = N - NS * RPW

_MESH = dict(core_axis_name="c", subcore_axis_name="s")


def _split_body(x_ref, o_ref):
    o_ref[...] = x_ref[...]


def _split(x):
    return pl.pallas_call(
        _split_body,
        grid=(NC, NB),
        in_specs=[pl.BlockSpec((RB, HALF), lambda c, i: (i, c))],
        out_specs=pl.BlockSpec((RB, HALF), lambda c, i: (c * NB + i, 0)),
        out_shape=jax.ShapeDtypeStruct((2 * N, HALF), jnp.float32),
    )(x)


def _z1_body(ha, hb, ga, gb, w, b, o):
    z = jnp.concatenate([ha[...] + ga[...], hb[...] + gb[...]], axis=1)
    o[...] = jnp.maximum(
        jnp.dot(z, w[...], preferred_element_type=jnp.float32) + b[...], 0.0)


def _z1(h2, agg2, W1, b1):
    return pl.pallas_call(
        _z1_body,
        grid=(NB,),
        in_specs=[
            pl.BlockSpec((RB, HALF), lambda i: (i, 0)),
            pl.BlockSpec((RB, HALF), lambda i: (i + NB, 0)),
            pl.BlockSpec((RB, HALF), lambda i: (i, 0)),
            pl.BlockSpec((RB, HALF), lambda i: (i + NB, 0)),
            pl.BlockSpec((D, D), lambda i: (0, 0)),
            pl.BlockSpec((1, D), lambda i: (0, 0)),
        ],
        out_specs=pl.BlockSpec((RB, D), lambda i: (i, 0)),
        out_shape=jax.ShapeDtypeStruct((N, D), jnp.float32),
    )(h2, h2, agg2, agg2, W1, b1.reshape(1, D))


def _z2_body(z, w, b, o, *, relu):
    r = jnp.dot(z[...], w[...], preferred_element_type=jnp.float32) + b[...]
    if relu:
        r = jnp.maximum(r, 0.0)
    o[...] = r


def _z2(z1, W2, b2, relu):
    return pl.pallas_call(
        functools.partial(_z2_body, relu=relu),
        grid=(NC, NB),
        in_specs=[
            pl.BlockSpec((RB, D), lambda c, i: (i, 0)),
            pl.BlockSpec((D, HALF), lambda c, i: (0, c)),
            pl.BlockSpec((1, HALF), lambda c, i: (0, c)),
        ],
        out_specs=pl.BlockSpec((RB, HALF), lambda c, i: (c * NB + i, 0)),
        out_shape=jax.ShapeDtypeStruct((2 * N, HALF), jnp.float32),
    )(z1, W2, b2.reshape(1, D))


def _score_body(ha, hb, ga, gb, w1, b1, w2, b2, o):
    z = jnp.concatenate([ha[...] + ga[...], hb[...] + gb[...]], axis=1)
    z1 = jnp.maximum(
        jnp.dot(z, w1[...], preferred_element_type=jnp.float32) + b1[...],
        0.0)
    s = jax.nn.sigmoid(
        jnp.dot(z1, w2[...], preferred_element_type=jnp.float32) + b2[...])
    o[...] = jax.lax.bitcast_convert_type(s, jnp.int32)


def _score(h2, agg2, sg_W1, sg_b1, sg_W2, sg_b2):
    return pl.pallas_call(
        _score_body,
        grid=(NB,),
        in_specs=[
            pl.BlockSpec((RB, HALF), lambda i: (i, 0)),
            pl.BlockSpec((RB, HALF), lambda i: (i + NB, 0)),
            pl.BlockSpec((RB, HALF), lambda i: (i, 0)),
            pl.BlockSpec((RB, HALF), lambda i: (i + NB, 0)),
            pl.BlockSpec((D, D), lambda i: (0, 0)),
            pl.BlockSpec((1, D), lambda i: (0, 0)),
            pl.BlockSpec((D, 1), lambda i: (0, 0)),
            pl.BlockSpec((1, 1), lambda i: (0, 0)),
        ],
        out_specs=pl.BlockSpec((RB, 1), lambda i: (i, 0)),
        out_shape=jax.ShapeDtypeStruct((N, 1), jnp.int32),
    )(h2, h2, agg2, agg2, sg_W1, sg_b1.reshape(1, D), sg_W2,
      sg_b2.reshape(1, 1))


def _maskx_body(xa, m, mn, tok, o):
    mm = m[...]
    o[...] = xa[...] * (1.0 - mm) + (mm - mn[...]) * tok[...]


def _maskx(x2, m, mn, token):
    return pl.pallas_call(
        _maskx_body,
        grid=(NC, NB),
        in_specs=[
            pl.BlockSpec((RB, HALF), lambda c, i: (c * NB + i, 0)),
            pl.BlockSpec((RB, 1), lambda c, i: (i, 0)),
            pl.BlockSpec((RB, 1), lambda c, i: (i, 0)),
            pl.BlockSpec((1, HALF), lambda c, i: (0, c)),
        ],
        out_specs=pl.BlockSpec((RB, HALF), lambda c, i: (c * NB + i, 0)),
        out_shape=jax.ShapeDtypeStruct((2 * N, HALF), jnp.float32),
    )(x2, m, mn, token)


def _rep_body(ha, hb, w, m, o):
    z = jnp.concatenate([ha[...], hb[...]], axis=1)
    o[...] = jnp.dot(z, w[...], preferred_element_type=jnp.float32) * (1.0 - m[...])


def _rep(h2, e2d_W, m):
    return pl.pallas_call(
        _rep_body,
        grid=(NC, NB),
        in_specs=[
            pl.BlockSpec((RB, HALF), lambda c, i: (i, 0)),
            pl.BlockSpec((RB, HALF), lambda c, i: (i + NB, 0)),
            pl.BlockSpec((D, HALF), lambda c, i: (0, c)),
            pl.BlockSpec((RB, 1), lambda c, i: (i, 0)),
        ],
        out_specs=pl.BlockSpec((RB, HALF), lambda c, i: (c * NB + i, 0)),
        out_shape=jax.ShapeDtypeStruct((2 * N, HALF), jnp.float32),
    )(h2, h2, e2d_W, m)


def _loss_body(ra, rb, xf, m, o):
    r = jnp.concatenate([ra[...], rb[...]], axis=1)
    t = xf[...]
    rr = jnp.sum(r * r, axis=1, keepdims=True)
    tt = jnp.sum(t * t, axis=1, keepdims=True)
    rt = jnp.sum(r * t, axis=1, keepdims=True)
    cos = rt / ((jnp.sqrt(rr) + 1e-8) * (jnp.sqrt(tt) + 1e-8))
    dd = 1.0 - cos
    v = jnp.sum(dd * dd * m[...]) * (1.0 / K_MASK)

    @pl.when(pl.program_id(0) == 0)
    def _():
        o[...] = jnp.zeros((1, 1), jnp.float32)

    o[...] += jnp.reshape(v, (1, 1))


def _loss(recon2, x, m):
    return pl.pallas_call(
        _loss_body,
        grid=(NB,),
        in_specs=[
            pl.BlockSpec((RB, HALF), lambda i: (i, 0)),
            pl.BlockSpec((RB, HALF), lambda i: (i + NB, 0)),
            pl.BlockSpec((RB, D), lambda i: (i, 0)),
            pl.BlockSpec((RB, 1), lambda i: (i, 0)),
        ],
        out_specs=pl.BlockSpec((1, 1), lambda i: (0, 0)),
        out_shape=jax.ShapeDtypeStruct((1, 1), jnp.float32),
    )(recon2, recon2, x, m)


def _segsum(h2, src, dst, zsrc):
    mesh = plsc.VectorSubcoreMesh(**_MESH)

    @functools.partial(
        pl.kernel, mesh=mesh,
        compiler_params=pltpu.CompilerParams(needs_layout_passes=False),
        out_type=jax.ShapeDtypeStruct((2 * N, HALF), jnp.float32),
        scratch_types=[
            pltpu.VMEM((CHUNK,), jnp.int32),
            pltpu.VMEM((CHUNK,), jnp.int32),
            pltpu.VMEM((CHUNK,), jnp.int32),
            pltpu.VMEM((CHUNK,), jnp.int32),
            pltpu.VMEM((CHUNK,), jnp.int32),
            pltpu.VMEM((CHUNK,), jnp.int32),
            pltpu.VMEM((CHUNK, HALF), jnp.float32),
            pltpu.VMEM((TAIL,), jnp.int32),
            pltpu.VMEM((TAIL,), jnp.int32),
            pltpu.VMEM((TAIL,), jnp.int32),
            pltpu.VMEM((TAIL, HALF), jnp.float32),
            pltpu.VMEM_SHARED((N, HALF), jnp.float32),
            pltpu.SemaphoreType.DMA,
            pltpu.SemaphoreType.DMA,
            pltpu.SemaphoreType.DMA,
        ],
    )
    def k(h2_hbm, src_hbm, dst_hbm, zsrc_hbm, out_hbm,
          sidxa, aidxa, didxa, sidxb, aidxb, didxb, gbuf,
          tsidx, taidx, tdidx, tgbuf, acc, sem, isema, isemb):
        c = lax.axis_index("c")
        s = lax.axis_index("s")
        coff = c * N
        r0 = s * RPW
        e0 = s * EPW

        def fetch_idx(eoff, si, di, isem):
            pltpu.async_copy(src_hbm.at[pl.ds(eoff, CHUNK)], si, isem)
            pltpu.async_copy(dst_hbm.at[pl.ds(eoff, CHUNK)], di, isem)

        def wait_idx(si, di, isem):
            pltpu.make_async_copy(src_hbm.at[pl.ds(0, CHUNK)], si,
                                  isem).wait()
            pltpu.make_async_copy(src_hbm.at[pl.ds(0, CHUNK)], di,
                                  isem).wait()

        def gs_chunk(si, ai, di):
            for l in range(CHUNK // LANES):
                sl = pl.ds(l * LANES, LANES)
                ai[sl] = si[sl] + coff
            pltpu.async_copy(h2_hbm.at[ai], gbuf, sem).wait()
            pltpu.sync_copy(gbuf, acc.at[di], add=True)

        fetch_idx(e0, sidxa, didxa, isema)

        pltpu.sync_copy(zsrc_hbm.at[pl.ds(0, RPW)], acc.at[pl.ds(r0, RPW)])

        @pl.when(s == NS - 1)
        def _():
            pltpu.sync_copy(zsrc_hbm.at[pl.ds(0, RTAIL)],
                            acc.at[pl.ds(NS * RPW, RTAIL)])

        plsc.subcore_barrier()

        def body(g, carry):
            j0 = 2 * g
            fetch_idx(e0 + (j0 + 1) * CHUNK, sidxb, didxb, isemb)
            wait_idx(sidxa, didxa, isema)
            gs_chunk(sidxa, aidxa, didxa)

            @pl.when(g < NFULL // 2 - 1)
            def _():
                fetch_idx(e0 + (j0 + 2) * CHUNK, sidxa, didxa, isema)

            wait_idx(sidxb, didxb, isemb)
            gs_chunk(sidxb, aidxb, didxb)
            return carry

        lax.fori_loop(0, NFULL // 2, body, 0)

        if TAIL:
            pltpu.sync_copy(src_hbm.at[pl.ds(e0 + NFULL * CHUNK, TAIL)],
                            tsidx)
            pltpu.sync_copy(dst_hbm.at[pl.ds(e0 + NFULL * CHUNK, TAIL)],
                            tdidx)
            for l in range(TAIL // LANES):
                sl = pl.ds(l * LANES, LANES)
                taidx[sl] = tsidx[sl] + coff
            pltpu.async_copy(h2_hbm.at[taidx], tgbuf, sem).wait()
            pltpu.sync_copy(tgbuf, acc.at[tdidx], add=True)

        plsc.subcore_barrier()
        pltpu.sync_copy(acc.at[pl.ds(r0, RPW)],
                        out_hbm.at[pl.ds(coff + r0, RPW)])

        @pl.when(s == NS - 1)
        def _():
            pltpu.sync_copy(acc.at[pl.ds(NS * RPW, RTAIL)],
                            out_hbm.at[pl.ds(coff + NS * RPW, RTAIL)])

    return k(h2, src, dst, zsrc)


def _select(scores):
    mesh = plsc.VectorSubcoreMesh(**_MESH)
    NT = NPAD // LANES

    @functools.partial(
        pl.kernel, mesh=mesh,
        compiler_params=pltpu.CompilerParams(needs_layout_passes=False),
        out_type=(jax.ShapeDtypeStruct((N,), jnp.float32),
                  jax.ShapeDtypeStruct((N,), jnp.float32),
                  jax.ShapeDtypeStruct((NPAD,), jnp.int32)),
        scratch_types=[
            pltpu.VMEM((N,), jnp.int32),
            pltpu.VMEM((N,), jnp.float32),
            pltpu.VMEM((N,), jnp.float32),
            pltpu.VMEM((176,), jnp.int32),
            pltpu.VMEM((176,), jnp.int32),
            pltpu.VMEM((NPAD,), jnp.int32),
            pltpu.VMEM((1504,), jnp.int32),
            pltpu.SemaphoreType.DMA,
        ],
    )
    def k(sc_hbm, m_hbm, mn_hbm, nd_hbm,
          kbuf, mbuf, mnbuf, tk, ti, nd, t15, sem):
        c = lax.axis_index("c")
        s = lax.axis_index("s")

        @pl.when(jnp.logical_and(c == 0, s == 0))
        def _():
            pltpu.sync_copy(sc_hbm, kbuf)

            one16 = jnp.ones((LANES,), jnp.int32)
            zero16i = jnp.zeros((LANES,), jnp.int32)

            def count_ge(t):
                def cb(j, acc):
                    v = kbuf[pl.ds(j * LANES, LANES)]
                    return acc + jnp.where(v >= t, one16, zero16i)
                accv = lax.fori_loop(0, NV, cb, jnp.zeros((LANES,), jnp.int32))
                return jnp.sum(accv)

            def bisect(kk, counter):
                def bb(it, lh):
                    lo, hi = lh
                    mid = lo + ((hi - lo) >> 1)
                    big = counter(mid) >= kk
                    return (jnp.where(big, mid, lo), jnp.where(big, hi, mid))
                lo, _ = lax.fori_loop(
                    0, 32, bb, (jnp.int32(0), jnp.int32(0x7FFFFFFF)))
                return lo

            t1500 = bisect(K_MASK, count_ge)

            zero16 = jnp.zeros((LANES,), jnp.int32)
            NV15 = 1504 // LANES

            def zt(j, carry):
                t15[pl.ds(j * LANES, LANES)] = zero16
                return carry
            lax.fori_loop(0, NV15, zt, 0)

            def comp(j, off):
                v = kbuf[pl.ds(j * LANES, LANES)]
                m1 = v >= t1500
                pc = plsc.cumsum(jnp.where(m1, one16, zero16i))
                plsc.store_scatter(t15, [off + pc - 1], v, mask=m1)
                return off + jnp.max(pc)
            lax.fori_loop(0, NV, comp, jnp.int32(0))

            def count_ge15(t):
                def cb(j, acc):
                    v = t15[pl.ds(j * LANES, LANES)]
                    return acc + jnp.where(v >= t, one16, zero16i)
                accv = lax.fori_loop(0, NV15, cb,
                                     jnp.zeros((LANES,), jnp.int32))
                return jnp.sum(accv)

            t150 = bisect(K_NOISE, count_ge15)
            for j in range(176 // LANES):
                tk[pl.ds(j * LANES, LANES)] = zero16

            iota16 = lax.iota(jnp.int32, LANES)

            def mb(j, off):
                sl = pl.ds(j * LANES, LANES)
                v = kbuf[sl]
                m1 = v >= t1500
                m2 = v >= t150
                mbuf[sl] = jnp.where(m1, 1.0, 0.0)
                mnbuf[sl] = jnp.where(m2, 1.0, 0.0)
                pc = plsc.cumsum(jnp.where(m2, one16, zero16i))
                pos = off + pc - 1
                ids = j * LANES + iota16
                plsc.store_scatter(tk, [pos], v, mask=m2)
                plsc.store_scatter(ti, [pos], ids, mask=m2)
                return off + jnp.max(pc)
            lax.fori_loop(0, NV, mb, jnp.int32(0))

            lane0 = iota16 == 0

            def rb(i, carry):
                iv = jnp.full((LANES,), i, jnp.int32)
                kv = plsc.load_gather(tk, [iv])

                def cb2(j, a):
                    w = tk[pl.ds(j * LANES, LANES)]
                    return a + jnp.where(w > kv, one16, zero16i)
                av = lax.fori_loop(0, NT, cb2, jnp.zeros((LANES,), jnp.int32))
                rv = jnp.full((LANES,), jnp.sum(av), jnp.int32)
                idv = plsc.load_gather(ti, [iv])
                plsc.store_scatter(nd, [rv], idv, mask=lane0)
                return carry
            lax.fori_loop(0, K_NOISE, rb, 0)

            nd0 = plsc.load_gather(nd, [jnp.zeros((LANES,), jnp.int32)])
            plsc.store_scatter(nd, [K_NOISE + iota16], nd0,
                               mask=iota16 < (NPAD - K_NOISE))

            pltpu.sync_copy(mbuf, m_hbm)
            pltpu.sync_copy(mnbuf, mn_hbm)
            pltpu.sync_copy(nd, nd_hbm)

    return k(scores)


def _noise_patch(base, x2, nsrc, nd):
    mesh = plsc.VectorSubcoreMesh(**_MESH)
    ROWS = 624
    CTAIL = 2 * N - NC * NS * ROWS

    @functools.partial(
        pl.kernel, mesh=mesh,
        compiler_params=pltpu.CompilerParams(needs_layout_passes=False),
        out_type=jax.ShapeDtypeStruct((2 * N, HALF), jnp.float32),
        scratch_types=[
            pltpu.VMEM((ROWS, HALF), jnp.float32),
            pltpu.VMEM((CTAIL, HALF), jnp.float32),
            pltpu.VMEM((NPAD,), jnp.int32),
            pltpu.VMEM((NPAD,), jnp.int32),
            pltpu.VMEM((NPAD, HALF), jnp.float32),
            pltpu.SemaphoreType.DMA,
        ],
    )
    def k(base_hbm, x2_hbm, nsrc_hbm, nd_hbm, out_hbm, buf, tbuf, gi, si,
          nbuf, sem):
        c = lax.axis_index("c")
        s = lax.axis_index("s")
        wid = c * NS + s
        r0 = wid * ROWS
        pltpu.sync_copy(base_hbm.at[pl.ds(r0, ROWS)], buf)
        pltpu.sync_copy(buf, out_hbm.at[pl.ds(r0, ROWS)])

        @pl.when(wid == NC * NS - 1)
        def _():
            t0 = NC * NS * ROWS
            pltpu.sync_copy(base_hbm.at[pl.ds(t0, CTAIL)], tbuf)
            pltpu.sync_copy(tbuf, out_hbm.at[pl.ds(t0, CTAIL)])

        plsc.subcore_barrier()

        @pl.when(s == 0)
        def _():
            coff = c * N
            pltpu.sync_copy(nsrc_hbm, gi)
            pltpu.sync_copy(nd_hbm, si)
            for l in range(NPAD // LANES):
                sl = pl.ds(l * LANES, LANES)
                gi[sl] = gi[sl] + coff
                si[sl] = si[sl] + coff
            pltpu.async_copy(x2_hbm.at[gi], nbuf, sem).wait()
            pltpu.sync_copy(nbuf, out_hbm.at[si])

    return k(base, x2, nsrc, nd)


def kernel(x, edge_index, epoch, max_epoch, sg_W1, sg_b1, sg_W2, sg_b2,
           sg2_W, sg2_b, e1_W1, e1_b1, e1_W2, e1_b2, e2_W1, e2_b1, e2_W2,
           e2_b2, e2d_W, d_W1, d_b1, d_W2, d_b2, enc_mask_token):
    src = edge_index[0]
    dst = edge_index[1]
    zsrc = jnp.zeros((RPW, HALF), jnp.float32)

    x2 = _split(x)

    agg0 = _segsum(x2, src, dst, zsrc)
    scores = _score(x2, agg0, sg_W1, sg_b1, sg_W2, sg_b2)

    m, mn, nd = _select(scores.reshape(N))
    m = m.reshape(N, 1)
    mn = mn.reshape(N, 1)

    nsrc = jax.random.randint(jax.random.key(7), (K_NOISE,), 0, N,
                              dtype=jnp.int32)
    nsrc = jnp.concatenate([nsrc, jnp.full((NPAD - K_NOISE,), nsrc[0],
                                           jnp.int32)])

    base = _maskx(x2, m, mn, enc_mask_token)
    out_x2 = _noise_patch(base, x2, nsrc, nd)

    agg1 = _segsum(out_x2, src, dst, zsrc)
    h = _z2(_z1(out_x2, agg1, e1_W1, e1_b1), e1_W2, e1_b2, relu=True)
    agg2 = _segsum(h, src, dst, zsrc)
    h = _z2(_z1(h, agg2, e2_W1, e2_b1), e2_W2, e2_b2, relu=True)

    rep = _rep(h, e2d_W, m)

    agg3 = _segsum(rep, src, dst, zsrc)
    recon = _z2(_z1(rep, agg3, d_W1, d_b1), d_W2, d_b2, relu=False)

    lossv = _loss(recon, x, m)[0, 0]
    ep = jnp.asarray(epoch).astype(jnp.float32)
    mep = jnp.asarray(max_epoch).astype(jnp.float32)
    return lossv + 0.0 * (0.3 * ep / mep)


# speedup vs baseline: 2.1320x; 1.2378x over previous
import functools

import jax
import jax.numpy as jnp
from jax import lax
from jax.experimental import pallas as pl
from jax.experimental.pallas import tpu as pltpu
from jax.experimental.pallas import tpu_sc as plsc

N = 10000
E = 160000
D = 256
HALF = 128
NB = 10
RB = N // NB
NC, NS, LANES = 2, 16, 16
K_MASK = 1500
K_NOISE = 150
NPAD = 160
EPW = E // NS
CHUNK = 128
NFULL = EPW // CHUNK
TAIL = EPW - NFULL * CHUNK
NV = N // LANES
RPW = 624
RTAIL = N - NS * RPW

_MESH = dict(core_axis_name="c", subcore_axis_name="s")


def _split_body(x_ref, o_ref):
    o_ref[...] = x_ref[...]


def _split(x):
    return pl.pallas_call(
        _split_body,
        grid=(NC, NB),
        in_specs=[pl.BlockSpec((RB, HALF), lambda c, i: (i, c))],
        out_specs=pl.BlockSpec((RB, HALF), lambda c, i: (c * NB + i, 0)),
        out_shape=jax.ShapeDtypeStruct((2 * N, HALF), jnp.float32),
    )(x)


def _z1_body(ha, hb, ga, gb, w, b, o):
    z = jnp.concatenate([ha[...] + ga[...], hb[...] + gb[...]], axis=1)
    o[...] = jnp.maximum(
        jnp.dot(z, w[...], preferred_element_type=jnp.float32) + b[...], 0.0)


def _z1(h2, agg2, W1, b1):
    return pl.pallas_call(
        _z1_body,
        grid=(NB,),
        in_specs=[
            pl.BlockSpec((RB, HALF), lambda i: (i, 0)),
            pl.BlockSpec((RB, HALF), lambda i: (i + NB, 0)),
            pl.BlockSpec((RB, HALF), lambda i: (i, 0)),
            pl.BlockSpec((RB, HALF), lambda i: (i + NB, 0)),
            pl.BlockSpec((D, D), lambda i: (0, 0)),
            pl.BlockSpec((1, D), lambda i: (0, 0)),
        ],
        out_specs=pl.BlockSpec((RB, D), lambda i: (i, 0)),
        out_shape=jax.ShapeDtypeStruct((N, D), jnp.float32),
    )(h2, h2, agg2, agg2, W1, b1.reshape(1, D))


def _z2_body(z, w, b, o, *, relu):
    r = jnp.dot(z[...], w[...], preferred_element_type=jnp.float32) + b[...]
    if relu:
        r = jnp.maximum(r, 0.0)
    o[...] = r


def _z2(z1, W2, b2, relu):
    return pl.pallas_call(
        functools.partial(_z2_body, relu=relu),
        grid=(NC, NB),
        in_specs=[
            pl.BlockSpec((RB, D), lambda c, i: (i, 0)),
            pl.BlockSpec((D, HALF), lambda c, i: (0, c)),
            pl.BlockSpec((1, HALF), lambda c, i: (0, c)),
        ],
        out_specs=pl.BlockSpec((RB, HALF), lambda c, i: (c * NB + i, 0)),
        out_shape=jax.ShapeDtypeStruct((2 * N, HALF), jnp.float32),
    )(z1, W2, b2.reshape(1, D))


def _score_body(ha, hb, ga, gb, w1, b1, w2, b2, o):
    z = jnp.concatenate([ha[...] + ga[...], hb[...] + gb[...]], axis=1)
    z1 = jnp.maximum(
        jnp.dot(z, w1[...], preferred_element_type=jnp.float32) + b1[...],
        0.0)
    s = jax.nn.sigmoid(
        jnp.dot(z1, w2[...], preferred_element_type=jnp.float32) + b2[...])
    o[...] = jax.lax.bitcast_convert_type(s, jnp.int32)


def _score(h2, agg2, sg_W1, sg_b1, sg_W2, sg_b2):
    return pl.pallas_call(
        _score_body,
        grid=(NB,),
        in_specs=[
            pl.BlockSpec((RB, HALF), lambda i: (i, 0)),
            pl.BlockSpec((RB, HALF), lambda i: (i + NB, 0)),
            pl.BlockSpec((RB, HALF), lambda i: (i, 0)),
            pl.BlockSpec((RB, HALF), lambda i: (i + NB, 0)),
            pl.BlockSpec((D, D), lambda i: (0, 0)),
            pl.BlockSpec((1, D), lambda i: (0, 0)),
            pl.BlockSpec((D, 1), lambda i: (0, 0)),
            pl.BlockSpec((1, 1), lambda i: (0, 0)),
        ],
        out_specs=pl.BlockSpec((RB, 1), lambda i: (i, 0)),
        out_shape=jax.ShapeDtypeStruct((N, 1), jnp.int32),
    )(h2, h2, agg2, agg2, sg_W1, sg_b1.reshape(1, D), sg_W2,
      sg_b2.reshape(1, 1))


def _maskx_body(xa, m, mn, tok, o):
    mm = m[...]
    o[...] = xa[...] * (1.0 - mm) + (mm - mn[...]) * tok[...]


def _maskx(x2, m, mn, token):
    return pl.pallas_call(
        _maskx_body,
        grid=(NC, NB),
        in_specs=[
            pl.BlockSpec((RB, HALF), lambda c, i: (c * NB + i, 0)),
            pl.BlockSpec((RB, 1), lambda c, i: (i, 0)),
            pl.BlockSpec((RB, 1), lambda c, i: (i, 0)),
            pl.BlockSpec((1, HALF), lambda c, i: (0, c)),
        ],
        out_specs=pl.BlockSpec((RB, HALF), lambda c, i: (c * NB + i, 0)),
        out_shape=jax.ShapeDtypeStruct((2 * N, HALF), jnp.float32),
    )(x2, m, mn, token)


def _rep_body(ha, hb, w, m, o):
    z = jnp.concatenate([ha[...], hb[...]], axis=1)
    o[...] = jnp.dot(z, w[...], preferred_element_type=jnp.float32) * (1.0 - m[...])


def _rep(h2, e2d_W, m):
    return pl.pallas_call(
        _rep_body,
        grid=(NC, NB),
        in_specs=[
            pl.BlockSpec((RB, HALF), lambda c, i: (i, 0)),
            pl.BlockSpec((RB, HALF), lambda c, i: (i + NB, 0)),
            pl.BlockSpec((D, HALF), lambda c, i: (0, c)),
            pl.BlockSpec((RB, 1), lambda c, i: (i, 0)),
        ],
        out_specs=pl.BlockSpec((RB, HALF), lambda c, i: (c * NB + i, 0)),
        out_shape=jax.ShapeDtypeStruct((2 * N, HALF), jnp.float32),
    )(h2, h2, e2d_W, m)


def _loss_body(ra, rb, xf, m, o):
    r = jnp.concatenate([ra[...], rb[...]], axis=1)
    t = xf[...]
    rr = jnp.sum(r * r, axis=1, keepdims=True)
    tt = jnp.sum(t * t, axis=1, keepdims=True)
    rt = jnp.sum(r * t, axis=1, keepdims=True)
    cos = rt / ((jnp.sqrt(rr) + 1e-8) * (jnp.sqrt(tt) + 1e-8))
    dd = 1.0 - cos
    v = jnp.sum(dd * dd * m[...]) * (1.0 / K_MASK)

    @pl.when(pl.program_id(0) == 0)
    def _():
        o[...] = jnp.zeros((1, 1), jnp.float32)

    o[...] += jnp.reshape(v, (1, 1))


def _loss(recon2, x, m):
    return pl.pallas_call(
        _loss_body,
        grid=(NB,),
        in_specs=[
            pl.BlockSpec((RB, HALF), lambda i: (i, 0)),
            pl.BlockSpec((RB, HALF), lambda i: (i + NB, 0)),
            pl.BlockSpec((RB, D), lambda i: (i, 0)),
            pl.BlockSpec((RB, 1), lambda i: (i, 0)),
        ],
        out_specs=pl.BlockSpec((1, 1), lambda i: (0, 0)),
        out_shape=jax.ShapeDtypeStruct((1, 1), jnp.float32),
    )(recon2, recon2, x, m)


def _segsum(h2, src, dst, zsrc):
    mesh = plsc.VectorSubcoreMesh(**_MESH)

    @functools.partial(
        pl.kernel, mesh=mesh,
        compiler_params=pltpu.CompilerParams(needs_layout_passes=False),
        out_type=jax.ShapeDtypeStruct((2 * N, HALF), jnp.float32),
        scratch_types=[
            pltpu.VMEM((CHUNK,), jnp.int32),
            pltpu.VMEM((CHUNK,), jnp.int32),
            pltpu.VMEM((CHUNK,), jnp.int32),
            pltpu.VMEM((CHUNK,), jnp.int32),
            pltpu.VMEM((CHUNK,), jnp.int32),
            pltpu.VMEM((CHUNK,), jnp.int32),
            pltpu.VMEM((CHUNK, HALF), jnp.float32),
            pltpu.VMEM((CHUNK, HALF), jnp.float32),
            pltpu.VMEM((TAIL,), jnp.int32),
            pltpu.VMEM((TAIL,), jnp.int32),
            pltpu.VMEM((TAIL,), jnp.int32),
            pltpu.VMEM((TAIL, HALF), jnp.float32),
            pltpu.VMEM_SHARED((N, HALF), jnp.float32),
            pltpu.SemaphoreType.DMA,
            pltpu.SemaphoreType.DMA,
            pltpu.SemaphoreType.DMA,
            pltpu.SemaphoreType.DMA,
        ],
    )
    def k(h2_hbm, src_hbm, dst_hbm, zsrc_hbm, out_hbm,
          sidxa, aidxa, didxa, sidxb, aidxb, didxb, gbufa, gbufb,
          tsidx, taidx, tdidx, tgbuf, acc, gsema, gsemb, isema, isemb):
        c = lax.axis_index("c")
        s = lax.axis_index("s")
        coff = c * N
        r0 = s * RPW
        e0 = s * EPW

        def fetch_idx(eoff, si, di, isem):
            pltpu.async_copy(src_hbm.at[pl.ds(eoff, CHUNK)], si, isem)
            pltpu.async_copy(dst_hbm.at[pl.ds(eoff, CHUNK)], di, isem)

        def wait_idx(si, di, isem):
            pltpu.make_async_copy(src_hbm.at[pl.ds(0, CHUNK)], si,
                                  isem).wait()
            pltpu.make_async_copy(src_hbm.at[pl.ds(0, CHUNK)], di,
                                  isem).wait()

        def adjust(si, ai):
            for l in range(CHUNK // LANES):
                sl = pl.ds(l * LANES, LANES)
                ai[sl] = si[sl] + coff

        fetch_idx(e0, sidxa, didxa, isema)

        pltpu.sync_copy(zsrc_hbm.at[pl.ds(0, RPW)], acc.at[pl.ds(r0, RPW)])

        @pl.when(s == NS - 1)
        def _():
            pltpu.sync_copy(zsrc_hbm.at[pl.ds(0, RTAIL)],
                            acc.at[pl.ds(NS * RPW, RTAIL)])

        plsc.subcore_barrier()

        wait_idx(sidxa, didxa, isema)
        adjust(sidxa, aidxa)
        pltpu.async_copy(h2_hbm.at[aidxa], gbufa, gsema)

        def body(g, carry):
            j0 = 2 * g
            fetch_idx(e0 + (j0 + 1) * CHUNK, sidxb, didxb, isemb)
            wait_idx(sidxb, didxb, isemb)
            adjust(sidxb, aidxb)
            pltpu.make_async_copy(h2_hbm, gbufa, gsema).wait()
            pltpu.async_copy(h2_hbm.at[aidxb], gbufb, gsemb)
            pltpu.sync_copy(gbufa, acc.at[didxa], add=True)

            @pl.when(g < NFULL // 2 - 1)
            def _():
                fetch_idx(e0 + (j0 + 2) * CHUNK, sidxa, didxa, isema)
                wait_idx(sidxa, didxa, isema)
                adjust(sidxa, aidxa)
                pltpu.async_copy(h2_hbm.at[aidxa], gbufa, gsema)

            pltpu.make_async_copy(h2_hbm, gbufb, gsemb).wait()
            pltpu.sync_copy(gbufb, acc.at[didxb], add=True)
            return carry

        lax.fori_loop(0, NFULL // 2, body, 0)

        if TAIL:
            pltpu.sync_copy(src_hbm.at[pl.ds(e0 + NFULL * CHUNK, TAIL)],
                            tsidx)
            pltpu.sync_copy(dst_hbm.at[pl.ds(e0 + NFULL * CHUNK, TAIL)],
                            tdidx)
            for l in range(TAIL // LANES):
                sl = pl.ds(l * LANES, LANES)
                taidx[sl] = tsidx[sl] + coff
            pltpu.async_copy(h2_hbm.at[taidx], tgbuf, gsema).wait()
            pltpu.sync_copy(tgbuf, acc.at[tdidx], add=True)

        plsc.subcore_barrier()
        pltpu.sync_copy(acc.at[pl.ds(r0, RPW)],
                        out_hbm.at[pl.ds(coff + r0, RPW)])

        @pl.when(s == NS - 1)
        def _():
            pltpu.sync_copy(acc.at[pl.ds(NS * RPW, RTAIL)],
                            out_hbm.at[pl.ds(coff + NS * RPW, RTAIL)])

    return k(h2, src, dst, zsrc)


def _select(scores):
    mesh = plsc.VectorSubcoreMesh(**_MESH)
    NT = NPAD // LANES

    @functools.partial(
        pl.kernel, mesh=mesh,
        compiler_params=pltpu.CompilerParams(needs_layout_passes=False),
        out_type=(jax.ShapeDtypeStruct((N,), jnp.float32),
                  jax.ShapeDtypeStruct((N,), jnp.float32),
                  jax.ShapeDtypeStruct((NPAD,), jnp.int32)),
        scratch_types=[
            pltpu.VMEM((N,), jnp.int32),
            pltpu.VMEM((N,), jnp.float32),
            pltpu.VMEM((N,), jnp.float32),
            pltpu.VMEM((176,), jnp.int32),
            pltpu.VMEM((176,), jnp.int32),
            pltpu.VMEM((NPAD,), jnp.int32),
            pltpu.VMEM((1504,), jnp.int32),
            pltpu.SemaphoreType.DMA,
        ],
    )
    def k(sc_hbm, m_hbm, mn_hbm, nd_hbm,
          kbuf, mbuf, mnbuf, tk, ti, nd, t15, sem):
        c = lax.axis_index("c")
        s = lax.axis_index("s")

        @pl.when(jnp.logical_and(c == 0, s == 0))
        def _():
            pltpu.sync_copy(sc_hbm, kbuf)

            one16 = jnp.ones((LANES,), jnp.int32)
            zero16i = jnp.zeros((LANES,), jnp.int32)

            def count_ge(t):
                def cb(j, acc):
                    v = kbuf[pl.ds(j * LANES, LANES)]
                    return acc + jnp.where(v >= t, one16, zero16i)
                accv = lax.fori_loop(0, NV, cb, jnp.zeros((LANES,), jnp.int32))
                return jnp.sum(accv)

            def bisect(kk, counter):
                def bb(it, lh):
                    lo, hi = lh
                    mid = lo + ((hi - lo) >> 1)
                    big = counter(mid) >= kk
                    return (jnp.where(big, mid, lo), jnp.where(big, hi, mid))
                lo, _ = lax.fori_loop(
                    0, 32, bb, (jnp.int32(0), jnp.int32(0x7FFFFFFF)))
                return lo

            t1500 = bisect(K_MASK, count_ge)

            zero16 = jnp.zeros((LANES,), jnp.int32)
            NV15 = 1504 // LANES

            def zt(j, carry):
                t15[pl.ds(j * LANES, LANES)] = zero16
                return carry
            lax.fori_loop(0, NV15, zt, 0)

            def comp(j, off):
                v = kbuf[pl.ds(j * LANES, LANES)]
                m1 = v >= t1500
                pc = plsc.cumsum(jnp.where(m1, one16, zero16i))
                plsc.store_scatter(t15, [off + pc - 1], v, mask=m1)
                return off + jnp.max(pc)
            lax.fori_loop(0, NV, comp, jnp.int32(0))

            def count_ge15(t):
                def cb(j, acc):
                    v = t15[pl.ds(j * LANES, LANES)]
                    return acc + jnp.where(v >= t, one16, zero16i)
                accv = lax.fori_loop(0, NV15, cb,
                                     jnp.zeros((LANES,), jnp.int32))
                return jnp.sum(accv)

            t150 = bisect(K_NOISE, count_ge15)
            for j in range(176 // LANES):
                tk[pl.ds(j * LANES, LANES)] = zero16

            iota16 = lax.iota(jnp.int32, LANES)

            def mb(j, off):
                sl = pl.ds(j * LANES, LANES)
                v = kbuf[sl]
                m1 = v >= t1500
                m2 = v >= t150
                mbuf[sl] = jnp.where(m1, 1.0, 0.0)
                mnbuf[sl] = jnp.where(m2, 1.0, 0.0)
                pc = plsc.cumsum(jnp.where(m2, one16, zero16i))
                pos = off + pc - 1
                ids = j * LANES + iota16
                plsc.store_scatter(tk, [pos], v, mask=m2)
                plsc.store_scatter(ti, [pos], ids, mask=m2)
                return off + jnp.max(pc)
            lax.fori_loop(0, NV, mb, jnp.int32(0))

            lane0 = iota16 == 0

            def rb(i, carry):
                iv = jnp.full((LANES,), i, jnp.int32)
                kv = plsc.load_gather(tk, [iv])

                def cb2(j, a):
                    w = tk[pl.ds(j * LANES, LANES)]
                    return a + jnp.where(w > kv, one16, zero16i)
                av = lax.fori_loop(0, NT, cb2, jnp.zeros((LANES,), jnp.int32))
                rv = jnp.full((LANES,), jnp.sum(av), jnp.int32)
                idv = plsc.load_gather(ti, [iv])
                plsc.store_scatter(nd, [rv], idv, mask=lane0)
                return carry
            lax.fori_loop(0, K_NOISE, rb, 0)

            nd0 = plsc.load_gather(nd, [jnp.zeros((LANES,), jnp.int32)])
            plsc.store_scatter(nd, [K_NOISE + iota16], nd0,
                               mask=iota16 < (NPAD - K_NOISE))

            pltpu.sync_copy(mbuf, m_hbm)
            pltpu.sync_copy(mnbuf, mn_hbm)
            pltpu.sync_copy(nd, nd_hbm)

    return k(scores)


def _noise_patch(base, x2, nsrc, nd):
    mesh = plsc.VectorSubcoreMesh(**_MESH)
    ROWS = 624
    CTAIL = 2 * N - NC * NS * ROWS

    @functools.partial(
        pl.kernel, mesh=mesh,
        compiler_params=pltpu.CompilerParams(needs_layout_passes=False),
        out_type=jax.ShapeDtypeStruct((2 * N, HALF), jnp.float32),
        scratch_types=[
            pltpu.VMEM((ROWS, HALF), jnp.float32),
            pltpu.VMEM((CTAIL, HALF), jnp.float32),
            pltpu.VMEM((NPAD,), jnp.int32),
            pltpu.VMEM((NPAD,), jnp.int32),
            pltpu.VMEM((NPAD, HALF), jnp.float32),
            pltpu.SemaphoreType.DMA,
        ],
    )
    def k(base_hbm, x2_hbm, nsrc_hbm, nd_hbm, out_hbm, buf, tbuf, gi, si,
          nbuf, sem):
        c = lax.axis_index("c")
        s = lax.axis_index("s")
        wid = c * NS + s
        r0 = wid * ROWS
        pltpu.sync_copy(base_hbm.at[pl.ds(r0, ROWS)], buf)
        pltpu.sync_copy(buf, out_hbm.at[pl.ds(r0, ROWS)])

        @pl.when(wid == NC * NS - 1)
        def _():
            t0 = NC * NS * ROWS
            pltpu.sync_copy(base_hbm.at[pl.ds(t0, CTAIL)], tbuf)
            pltpu.sync_copy(tbuf, out_hbm.at[pl.ds(t0, CTAIL)])

        plsc.subcore_barrier()

        @pl.when(s == 0)
        def _():
            coff = c * N
            pltpu.sync_copy(nsrc_hbm, gi)
            pltpu.sync_copy(nd_hbm, si)
            for l in range(NPAD // LANES):
                sl = pl.ds(l * LANES, LANES)
                gi[sl] = gi[sl] + coff
                si[sl] = si[sl] + coff
            pltpu.async_copy(x2_hbm.at[gi], nbuf, sem).wait()
            pltpu.sync_copy(nbuf, out_hbm.at[si])

    return k(base, x2, nsrc, nd)


def kernel(x, edge_index, epoch, max_epoch, sg_W1, sg_b1, sg_W2, sg_b2,
           sg2_W, sg2_b, e1_W1, e1_b1, e1_W2, e1_b2, e2_W1, e2_b1, e2_W2,
           e2_b2, e2d_W, d_W1, d_b1, d_W2, d_b2, enc_mask_token):
    src = edge_index[0]
    dst = edge_index[1]
    zsrc = jnp.zeros((RPW, HALF), jnp.float32)

    x2 = _split(x)

    agg0 = _segsum(x2, src, dst, zsrc)
    scores = _score(x2, agg0, sg_W1, sg_b1, sg_W2, sg_b2)

    m, mn, nd = _select(scores.reshape(N))
    m = m.reshape(N, 1)
    mn = mn.reshape(N, 1)

    nsrc = jax.random.randint(jax.random.key(7), (K_NOISE,), 0, N,
                              dtype=jnp.int32)
    nsrc = jnp.concatenate([nsrc, jnp.full((NPAD - K_NOISE,), nsrc[0],
                                           jnp.int32)])

    base = _maskx(x2, m, mn, enc_mask_token)
    out_x2 = _noise_patch(base, x2, nsrc, nd)

    agg1 = _segsum(out_x2, src, dst, zsrc)
    h = _z2(_z1(out_x2, agg1, e1_W1, e1_b1), e1_W2, e1_b2, relu=True)
    agg2 = _segsum(h, src, dst, zsrc)
    h = _z2(_z1(h, agg2, e2_W1, e2_b1), e2_W2, e2_b2, relu=True)

    rep = _rep(h, e2d_W, m)

    agg3 = _segsum(rep, src, dst, zsrc)
    recon = _z2(_z1(rep, agg3, d_W1, d_b1), d_W2, d_b2, relu=False)

    lossv = _loss(recon, x, m)[0, 0]
    ep = jnp.asarray(epoch).astype(jnp.float32)
    mep = jnp.asarray(max_epoch).astype(jnp.float32)
    return lossv + 0.0 * (0.3 * ep / mep)
